# Initial kernel scaffold; baseline (speedup 1.0000x reference)
import jax
import jax.numpy as jnp
from jax.experimental import pallas as pl


def kernel(x, pos, edge_index, W1, b1, W2, b2, W3, b3, W4, b4):
    raise NotImplementedError("write your pallas kernel here")


# speedup vs baseline: 2.7940x; 2.7940x over previous
import functools

import jax
import jax.numpy as jnp
from jax import lax
from jax.experimental import pallas as pl
from jax.experimental.pallas import tpu as pltpu
from jax.experimental.pallas import tpu_sc as plsc

NC = 2
NS = 16
NW = NC * NS
LANES = 16
CH = 128


def _node_tables(x, W1aT, W1bT, b1, W3T, b3, w4row, b4):
    N, D = x.shape
    H = W1aT.shape[1]
    BN = 1000
    assert N % BN == 0

    def body(x_ref, w1a_ref, w1b_ref, b1_ref, w3_ref, b3_ref, w4_ref, b4_ref,
             xa_ref, xbb_ref, wp_ref):
        xb = x_ref[...]
        xa_ref[...] = jnp.dot(xb, w1a_ref[...], preferred_element_type=jnp.float32)
        xbb_ref[...] = jnp.dot(xb, w1b_ref[...], preferred_element_type=jnp.float32) + b1_ref[...]
        h2 = jax.nn.silu(jnp.dot(xb, w3_ref[...], preferred_element_type=jnp.float32) + b3_ref[...])
        wp_ref[...] = jnp.sum(h2 * w4_ref[...], axis=1, keepdims=True) + b4_ref[...]

    return pl.pallas_call(
        body,
        grid=(N // BN,),
        in_specs=[
            pl.BlockSpec((BN, D), lambda i: (i, 0)),
            pl.BlockSpec((D, H), lambda i: (0, 0)),
            pl.BlockSpec((D, H), lambda i: (0, 0)),
            pl.BlockSpec((1, H), lambda i: (0, 0)),
            pl.BlockSpec((D, H), lambda i: (0, 0)),
            pl.BlockSpec((1, H), lambda i: (0, 0)),
            pl.BlockSpec((1, H), lambda i: (0, 0)),
            pl.BlockSpec((1, 1), lambda i: (0, 0)),
        ],
        out_specs=[
            pl.BlockSpec((BN, H), lambda i: (i, 0)),
            pl.BlockSpec((BN, H), lambda i: (i, 0)),
            pl.BlockSpec((BN, 1), lambda i: (i, 0)),
        ],
        out_shape=[
            jax.ShapeDtypeStruct((N, H), jnp.float32),
            jax.ShapeDtypeStruct((N, H), jnp.float32),
            jax.ShapeDtypeStruct((N, 1), jnp.float32),
        ],
    )(x, W1aT, W1bT, b1, W3T, b3, w4row, b4)


def _edge_gather_g(T, U, rowp, colg, cols, w1c, E_pad, NP, H):
    TW = T.shape[1]
    PWC = E_pad // (NW * CH)
    RPS = NP // NS
    NJ = H // LANES
    nfull, rem = RPS // CH, RPS % CH
    mesh = plsc.VectorSubcoreMesh(core_axis_name="c", subcore_axis_name="s")

    @functools.partial(
        pl.kernel,
        mesh=mesh,
        compiler_params=pltpu.CompilerParams(use_tc_tiling_on_sc=False),
        out_type=[
            jax.ShapeDtypeStruct((E_pad, H), jnp.float32),
            jax.ShapeDtypeStruct((NC, NP, LANES), jnp.float32),
        ],
        scratch_types=[
            pltpu.VMEM((CH,), jnp.int32),
            pltpu.VMEM((CH,), jnp.int32),
            pltpu.VMEM((CH,), jnp.int32),
            pltpu.VMEM((CH, TW), jnp.float32),
            pltpu.VMEM((CH, TW), jnp.float32),
            pltpu.VMEM((CH, H), jnp.float32),
            pltpu.VMEM((CH, LANES), jnp.float32),
            pltpu.VMEM((H,), jnp.float32),
            pltpu.VMEM_SHARED((NP, LANES), jnp.float32),
            pltpu.SemaphoreType.DMA,
            pltpu.SemaphoreType.DMA,
        ],
    )
    def k(t_hbm, u_hbm, rowp_hbm, colg_hbm, cols_hbm, w1c_hbm,
          g_hbm, pacc_hbm,
          idx_r, idx_cg, idx_cs, tbuf, ubuf, gbuf, pubuf, w1cv,
          shared_pos, sem0, sem1):
        c = lax.axis_index("c")
        s = lax.axis_index("s")
        wid = s * NC + c

        pltpu.sync_copy(w1c_hbm, w1cv)
        w1cs = [w1cv[pl.ds(LANES * j, LANES)] for j in range(NJ)]
        io = lax.iota(jnp.int32, LANES)
        mask3 = jnp.where(io < 3, 1.0, 0.0).astype(jnp.float32)

        zero16 = jnp.zeros((LANES,), jnp.float32)

        @pl.loop(0, CH)
        def _(r):
            pubuf[r, :] = zero16

        r0 = s * RPS
        for kk in range(nfull):
            pltpu.sync_copy(pubuf, shared_pos.at[pl.ds(r0 + CH * kk, CH)])
        if rem:
            pltpu.sync_copy(pubuf.at[pl.ds(0, rem)],
                            shared_pos.at[pl.ds(r0 + CH * nfull, rem)])
        plsc.subcore_barrier()

        base = wid * (PWC * CH)

        @pl.loop(0, PWC)
        def _(ch):
            e0 = base + ch * CH
            pltpu.sync_copy(rowp_hbm.at[pl.ds(e0, CH)], idx_r)
            pltpu.sync_copy(colg_hbm.at[pl.ds(e0, CH)], idx_cg)
            pltpu.sync_copy(cols_hbm.at[pl.ds(e0, CH)], idx_cs)
            cp1 = pltpu.async_copy(t_hbm.at[idx_r], tbuf, sem0)
            cp2 = pltpu.async_copy(u_hbm.at[idx_cg], ubuf, sem1)
            cp1.wait()
            cp2.wait()

            @pl.loop(0, CH)
            def _(e):
                t8 = tbuf[e, pl.ds(H, LANES)]
                u8 = ubuf[e, pl.ds(H, LANES)]
                r = t8 - u8
                rel = r * mask3
                d = r[0] * r[0] + r[1] * r[1] + r[2] * r[2]
                wp = t8[3]
                pubuf[e, :] = wp * rel
                for j in range(NJ):
                    sl = pl.ds(LANES * j, LANES)
                    gbuf[e, sl] = tbuf[e, sl] + ubuf[e, sl] + d * w1cs[j]

            pltpu.sync_copy(gbuf, g_hbm.at[pl.ds(e0, CH)])
            pltpu.sync_copy(pubuf, shared_pos.at[idx_cs], add=True)

        plsc.subcore_barrier()
        for kk in range(nfull):
            pltpu.sync_copy(shared_pos.at[pl.ds(r0 + CH * kk, CH)], pubuf)
            pltpu.sync_copy(pubuf, pacc_hbm.at[c, pl.ds(r0 + CH * kk, CH)])
        if rem:
            pltpu.sync_copy(shared_pos.at[pl.ds(r0 + CH * nfull, rem)],
                            pubuf.at[pl.ds(0, rem)])
            pltpu.sync_copy(pubuf.at[pl.ds(0, rem)],
                            pacc_hbm.at[c, pl.ds(r0 + CH * nfull, rem)])

    return k(T, U, rowp, colg, cols, w1c)


def _edge_mlp(g, W2T, b2):
    E_pad, H = g.shape
    D = W2T.shape[1]
    BE = 1024
    assert E_pad % BE == 0

    def body(g_ref, w2_ref, b2_ref, msg_ref):
        h = jax.nn.silu(g_ref[...])
        msg_ref[...] = jnp.dot(h, w2_ref[...], preferred_element_type=jnp.float32) + b2_ref[...]

    return pl.pallas_call(
        body,
        grid=(E_pad // BE,),
        in_specs=[
            pl.BlockSpec((BE, H), lambda i: (i, 0)),
            pl.BlockSpec((H, D), lambda i: (0, 0)),
            pl.BlockSpec((1, D), lambda i: (0, 0)),
        ],
        out_specs=pl.BlockSpec((BE, D), lambda i: (i, 0)),
        out_shape=jax.ShapeDtypeStruct((E_pad, D), jnp.float32),
    )(g, W2T, b2)


def _scatter_msg(msg, cols, NP):
    E_pad, D = msg.shape
    PWC = E_pad // (NW * CH)
    RPS = NP // NS
    nfull, rem = RPS // CH, RPS % CH
    mesh = plsc.VectorSubcoreMesh(core_axis_name="c", subcore_axis_name="s")

    @functools.partial(
        pl.kernel,
        mesh=mesh,
        compiler_params=pltpu.CompilerParams(use_tc_tiling_on_sc=False),
        out_type=jax.ShapeDtypeStruct((NC, NP, D), jnp.float32),
        scratch_types=[
            pltpu.VMEM((CH,), jnp.int32),
            pltpu.VMEM((CH, D), jnp.float32),
            pltpu.VMEM_SHARED((NP, D), jnp.float32),
            pltpu.SemaphoreType.DMA,
        ],
    )
    def k(msg_hbm, cols_hbm, xacc_hbm, idx_v, mbuf, shared_x, sem0):
        c = lax.axis_index("c")
        s = lax.axis_index("s")
        wid = s * NC + c
        zero16 = jnp.zeros((LANES,), jnp.float32)

        @pl.loop(0, CH)
        def _(r):
            for j in range(D // LANES):
                mbuf[r, pl.ds(LANES * j, LANES)] = zero16

        r0 = s * RPS
        for kk in range(nfull):
            pltpu.sync_copy(mbuf, shared_x.at[pl.ds(r0 + CH * kk, CH)])
        if rem:
            pltpu.sync_copy(mbuf.at[pl.ds(0, rem)],
                            shared_x.at[pl.ds(r0 + CH * nfull, rem)])
        plsc.subcore_barrier()

        base = wid * (PWC * CH)

        @pl.loop(0, PWC)
        def _(ch):
            e0 = base + ch * CH
            pltpu.sync_copy(cols_hbm.at[pl.ds(e0, CH)], idx_v)
            pltpu.async_copy(msg_hbm.at[pl.ds(e0, CH)], mbuf, sem0).wait()
            pltpu.sync_copy(mbuf, shared_x.at[idx_v], add=True)

        plsc.subcore_barrier()
        for kk in range(nfull):
            pltpu.sync_copy(shared_x.at[pl.ds(r0 + CH * kk, CH)], mbuf)
            pltpu.sync_copy(mbuf, xacc_hbm.at[c, pl.ds(r0 + CH * kk, CH)])
        if rem:
            pltpu.sync_copy(shared_x.at[pl.ds(r0 + CH * nfull, rem)],
                            mbuf.at[pl.ds(0, rem)])
            pltpu.sync_copy(mbuf.at[pl.ds(0, rem)],
                            xacc_hbm.at[c, pl.ds(r0 + CH * nfull, rem)])

    return k(msg, cols)


def _combine(xacc, pacc, N):
    _, NP, D = xacc.shape
    L = pacc.shape[2]
    BN = 1000
    assert N % BN == 0

    def body(x_ref, p_ref, ax_ref, ap_ref):
        ax_ref[...] = x_ref[0] + x_ref[1]
        ps = p_ref[0] + p_ref[1]
        ap_ref[...] = ps[:, :3]

    return pl.pallas_call(
        body,
        grid=(N // BN,),
        in_specs=[
            pl.BlockSpec((NC, BN, D), lambda i: (0, i, 0)),
            pl.BlockSpec((NC, BN, L), lambda i: (0, i, 0)),
        ],
        out_specs=[
            pl.BlockSpec((BN, D), lambda i: (i, 0)),
            pl.BlockSpec((BN, 3), lambda i: (i, 0)),
        ],
        out_shape=[
            jax.ShapeDtypeStruct((N, D), jnp.float32),
            jax.ShapeDtypeStruct((N, 3), jnp.float32),
        ],
    )(xacc, pacc)


def kernel(x, pos, edge_index, W1, b1, W2, b2, W3, b3, W4, b4):
    N, D = x.shape
    E = edge_index.shape[1]
    H = W1.shape[0]

    PWC = -(-E // (NW * CH))
    E_pad = NW * CH * PWC
    PAD = E_pad - E
    NDUM = 64
    NP = -(-(N + NDUM) // (NS * 8)) * (NS * 8)

    W1aT = W1[:, :D].T
    W1bT = W1[:, D:2 * D].T
    w1c = W1[:, 2 * D]
    W2T = W2.T
    W3T = W3.T
    b1r = b1.reshape(1, H)
    b2r = b2.reshape(1, D)
    b3r = b3.reshape(1, H)
    w4row = W4.reshape(1, H)
    b4r = b4.reshape(1, 1)

    xa, xbb, wp = _node_tables(x, W1aT, W1bT, b1r, W3T, b3r, w4row, b4r)

    zpadT = jnp.zeros((N, 12), jnp.float32)
    zpadU = jnp.zeros((N, 13), jnp.float32)
    T = jnp.concatenate([xa, pos, wp, zpadT], axis=1)
    U = jnp.concatenate([xbb, pos, zpadU], axis=1)

    row = edge_index[0]
    col = edge_index[1]
    rowp = jnp.concatenate([row, jnp.zeros((PAD,), jnp.int32)])
    colg = jnp.concatenate([col, jnp.zeros((PAD,), jnp.int32)])
    cols = jnp.concatenate(
        [col, (N + jnp.arange(PAD, dtype=jnp.int32) % NDUM)])

    g, pacc = _edge_gather_g(T, U, rowp, colg, cols, w1c, E_pad, NP, H)
    msg = _edge_mlp(g, W2T, b2r)
    xacc = _scatter_msg(msg, cols, NP)
    aggregated_x, aggregated_pos = _combine(xacc, pacc, N)
    return (aggregated_x, aggregated_pos)


# speedup vs baseline: 4.0757x; 1.4587x over previous
import functools

import jax
import jax.numpy as jnp
from jax import lax
from jax.experimental import pallas as pl
from jax.experimental.pallas import tpu as pltpu
from jax.experimental.pallas import tpu_sc as plsc

NC = 2
NS = 16
NW = NC * NS
LANES = 16
CH = 128


def _node_tables(x, W1aT, W1bT, b1, W3T, b3, w4row, b4):
    N, D = x.shape
    H = W1aT.shape[1]
    BN = 1000
    assert N % BN == 0

    def body(x_ref, w1a_ref, w1b_ref, b1_ref, w3_ref, b3_ref, w4_ref, b4_ref,
             xa_ref, xbb_ref, wp_ref):
        xb = x_ref[...]
        xa_ref[...] = jnp.dot(xb, w1a_ref[...], preferred_element_type=jnp.float32)
        xbb_ref[...] = jnp.dot(xb, w1b_ref[...], preferred_element_type=jnp.float32) + b1_ref[...]
        h2 = jax.nn.silu(jnp.dot(xb, w3_ref[...], preferred_element_type=jnp.float32) + b3_ref[...])
        wp_ref[...] = jnp.sum(h2 * w4_ref[...], axis=1, keepdims=True) + b4_ref[...]

    return pl.pallas_call(
        body,
        grid=(N // BN,),
        in_specs=[
            pl.BlockSpec((BN, D), lambda i: (i, 0)),
            pl.BlockSpec((D, H), lambda i: (0, 0)),
            pl.BlockSpec((D, H), lambda i: (0, 0)),
            pl.BlockSpec((1, H), lambda i: (0, 0)),
            pl.BlockSpec((D, H), lambda i: (0, 0)),
            pl.BlockSpec((1, H), lambda i: (0, 0)),
            pl.BlockSpec((1, H), lambda i: (0, 0)),
            pl.BlockSpec((1, 1), lambda i: (0, 0)),
        ],
        out_specs=[
            pl.BlockSpec((BN, H), lambda i: (i, 0)),
            pl.BlockSpec((BN, H), lambda i: (i, 0)),
            pl.BlockSpec((BN, 1), lambda i: (i, 0)),
        ],
        out_shape=[
            jax.ShapeDtypeStruct((N, H), jnp.float32),
            jax.ShapeDtypeStruct((N, H), jnp.float32),
            jax.ShapeDtypeStruct((N, 1), jnp.float32),
        ],
    )(x, W1aT, W1bT, b1, W3T, b3, w4row, b4)


def _edge_gather_g(T, U, idxcat, w1c, E_pad, NP, H):
    TW = T.shape[1]
    PWC = E_pad // (NW * CH)
    NPAIR = PWC // 2
    RPS = NP // NS
    NJ = H // LANES
    nfull, rem = RPS // CH, RPS % CH
    mesh = plsc.VectorSubcoreMesh(core_axis_name="c", subcore_axis_name="s")

    @functools.partial(
        pl.kernel,
        mesh=mesh,
        compiler_params=pltpu.CompilerParams(use_tc_tiling_on_sc=False),
        out_type=[
            jax.ShapeDtypeStruct((E_pad, H), jnp.float32),
            jax.ShapeDtypeStruct((NC, NP, LANES), jnp.float32),
        ],
        scratch_types=[
            pltpu.VMEM((2, 3, CH), jnp.int32),
            pltpu.VMEM((2, CH), jnp.int32),
            pltpu.VMEM((2, CH, TW), jnp.float32),
            pltpu.VMEM((2, CH, TW), jnp.float32),
            pltpu.VMEM((2, CH, H), jnp.float32),
            pltpu.VMEM((2, CH, LANES), jnp.float32),
            pltpu.VMEM((H,), jnp.float32),
            pltpu.VMEM_SHARED((NP, LANES), jnp.float32),
            pltpu.SemaphoreType.DMA,
            pltpu.SemaphoreType.DMA,
            pltpu.SemaphoreType.DMA,
            pltpu.SemaphoreType.DMA,
            pltpu.SemaphoreType.DMA,
            pltpu.SemaphoreType.DMA,
            pltpu.SemaphoreType.DMA,
            pltpu.SemaphoreType.DMA,
        ],
    )
    def k(t_hbm, u_hbm, idx_hbm, w1c_hbm,
          g_hbm, pacc_hbm,
          ibuf, sbuf, tbuf, ubuf, gbuf, pubuf, w1cv, shared_pos,
          isem0, isem1, gsm0, gsm1, stm0, stm1, scm0, scm1):
        c = lax.axis_index("c")
        s = lax.axis_index("s")
        wid = s * NC + c
        isem = (isem0, isem1)
        gsm = (gsm0, gsm1)
        stm = (stm0, stm1)
        scm = (scm0, scm1)

        pltpu.sync_copy(w1c_hbm, w1cv)
        w1cs = [w1cv[pl.ds(LANES * j, LANES)] for j in range(NJ)]
        io = lax.iota(jnp.int32, LANES)
        mask3 = jnp.where(io < 3, 1.0, 0.0).astype(jnp.float32)
        zero16 = jnp.zeros((LANES,), jnp.float32)

        @pl.loop(0, CH)
        def _(r):
            pubuf[0, r, :] = zero16

        r0 = s * RPS
        for kk in range(nfull):
            pltpu.sync_copy(pubuf.at[0], shared_pos.at[pl.ds(r0 + CH * kk, CH)])
        if rem:
            pltpu.sync_copy(pubuf.at[0, pl.ds(0, rem)],
                            shared_pos.at[pl.ds(r0 + CH * nfull, rem)])
        plsc.subcore_barrier()

        cid0 = wid * PWC

        def gathers(cid, b):
            cp1 = pltpu.async_copy(t_hbm.at[ibuf.at[b, 0]], tbuf.at[b], gsm[b])
            cp2 = pltpu.async_copy(u_hbm.at[ibuf.at[b, 1]], ubuf.at[b], gsm[b])
            return cp1, cp2

        pltpu.sync_copy(idx_hbm.at[cid0], ibuf.at[0])
        gathers(cid0, 0)
        pltpu.async_copy(idx_hbm.at[cid0 + 1], ibuf.at[1], isem1)

        @pl.loop(0, NPAIR)
        def _(p):
            for b in (0, 1):
                nb = 1 - b
                ch = 2 * p + b
                cid = cid0 + ch
                e0 = cid * CH

                pltpu.make_async_copy(t_hbm.at[ibuf.at[b, 0]], tbuf.at[b], gsm[b]).wait()
                pltpu.make_async_copy(u_hbm.at[ibuf.at[b, 1]], ubuf.at[b], gsm[b]).wait()

                @pl.when(p > 0)
                def _():
                    pltpu.make_async_copy(gbuf.at[b], g_hbm.at[pl.ds(e0, CH)], stm[b]).wait()
                    pltpu.make_async_copy(pubuf.at[b], shared_pos.at[sbuf.at[b]], scm[b]).wait()

                for j in range(CH // LANES):
                    sl = pl.ds(LANES * j, LANES)
                    sbuf[b, sl] = ibuf[b, 2, sl]

                @pl.when(p < NPAIR - 1)
                def _():
                    pltpu.async_copy(idx_hbm.at[cid + 2], ibuf.at[b], isem[b])

                def launch_next():
                    pltpu.make_async_copy(idx_hbm.at[cid], ibuf.at[nb], isem[nb]).wait()
                    gathers(cid + 1, nb)
                if b == 0:
                    launch_next()
                else:
                    pl.when(p < NPAIR - 1)(launch_next)

                tb = tbuf.at[b]
                ub = ubuf.at[b]
                gb = gbuf.at[b]
                pb = pubuf.at[b]

                @pl.loop(0, CH)
                def _(e):
                    t8 = tb[e, pl.ds(H, LANES)]
                    u8 = ub[e, pl.ds(H, LANES)]
                    r = t8 - u8
                    rel = r * mask3
                    d = r[0] * r[0] + r[1] * r[1] + r[2] * r[2]
                    wp = t8[3]
                    pb[e, :] = wp * rel
                    for j in range(NJ):
                        sl = pl.ds(LANES * j, LANES)
                        gb[e, sl] = tb[e, sl] + ub[e, sl] + d * w1cs[j]

                pltpu.async_copy(gbuf.at[b], g_hbm.at[pl.ds(e0, CH)], stm[b])
                pltpu.async_copy(pubuf.at[b], shared_pos.at[sbuf.at[b]], scm[b],
                                 add=True)

        for b in (0, 1):
            pltpu.make_async_copy(gbuf.at[b], g_hbm.at[pl.ds(0, CH)], stm[b]).wait()
            pltpu.make_async_copy(pubuf.at[b], shared_pos.at[sbuf.at[b]], scm[b]).wait()

        plsc.subcore_barrier()
        for kk in range(nfull):
            pltpu.sync_copy(shared_pos.at[pl.ds(r0 + CH * kk, CH)], pubuf.at[0])
            pltpu.sync_copy(pubuf.at[0], pacc_hbm.at[c, pl.ds(r0 + CH * kk, CH)])
        if rem:
            pltpu.sync_copy(shared_pos.at[pl.ds(r0 + CH * nfull, rem)],
                            pubuf.at[0, pl.ds(0, rem)])
            pltpu.sync_copy(pubuf.at[0, pl.ds(0, rem)],
                            pacc_hbm.at[c, pl.ds(r0 + CH * nfull, rem)])

    return k(T, U, idxcat, w1c)


def _edge_mlp(g, W2T, b2):
    E_pad, H = g.shape
    D = W2T.shape[1]
    BE = 1024
    assert E_pad % BE == 0

    def body(g_ref, w2_ref, b2_ref, msg_ref):
        h = jax.nn.silu(g_ref[...])
        msg_ref[...] = jnp.dot(h, w2_ref[...], preferred_element_type=jnp.float32) + b2_ref[...]

    return pl.pallas_call(
        body,
        grid=(E_pad // BE,),
        in_specs=[
            pl.BlockSpec((BE, H), lambda i: (i, 0)),
            pl.BlockSpec((H, D), lambda i: (0, 0)),
            pl.BlockSpec((1, D), lambda i: (0, 0)),
        ],
        out_specs=pl.BlockSpec((BE, D), lambda i: (i, 0)),
        out_shape=jax.ShapeDtypeStruct((E_pad, D), jnp.float32),
    )(g, W2T, b2)


def _scatter_msg(msg, idxcat, NP):
    E_pad, D = msg.shape
    PWC = E_pad // (NW * CH)
    assert PWC % 2 == 0
    NPAIR = PWC // 2
    RPS = NP // NS
    nfull, rem = RPS // CH, RPS % CH
    mesh = plsc.VectorSubcoreMesh(core_axis_name="c", subcore_axis_name="s")

    @functools.partial(
        pl.kernel,
        mesh=mesh,
        compiler_params=pltpu.CompilerParams(use_tc_tiling_on_sc=False),
        out_type=jax.ShapeDtypeStruct((NC, NP, D), jnp.float32),
        scratch_types=[
            pltpu.VMEM((2, CH), jnp.int32),
            pltpu.VMEM((2, CH, D), jnp.float32),
            pltpu.VMEM_SHARED((NP, D), jnp.float32),
            pltpu.SemaphoreType.DMA,
            pltpu.SemaphoreType.DMA,
            pltpu.SemaphoreType.DMA,
            pltpu.SemaphoreType.DMA,
        ],
    )
    def k(msg_hbm, idx_hbm, xacc_hbm, ibuf, mbuf, shared_x,
          lsem0, lsem1, ssem0, ssem1):
        c = lax.axis_index("c")
        s = lax.axis_index("s")
        wid = s * NC + c
        lsem = (lsem0, lsem1)
        ssem = (ssem0, ssem1)
        zero16 = jnp.zeros((LANES,), jnp.float32)

        @pl.loop(0, CH)
        def _(r):
            for j in range(D // LANES):
                mbuf[0, r, pl.ds(LANES * j, LANES)] = zero16

        r0 = s * RPS
        for kk in range(nfull):
            pltpu.sync_copy(mbuf.at[0], shared_x.at[pl.ds(r0 + CH * kk, CH)])
        if rem:
            pltpu.sync_copy(mbuf.at[0, pl.ds(0, rem)],
                            shared_x.at[pl.ds(r0 + CH * nfull, rem)])
        plsc.subcore_barrier()

        cid0 = wid * PWC

        def loads(cid, b):
            pltpu.async_copy(msg_hbm.at[pl.ds(cid * CH, CH)], mbuf.at[b], lsem[b])
            pltpu.async_copy(idx_hbm.at[cid, 2], ibuf.at[b], lsem[b])

        loads(cid0, 0)

        @pl.loop(0, NPAIR)
        def _(p):
            for b in (0, 1):
                nb = 1 - b
                ch = 2 * p + b
                cid = cid0 + ch

                def drain():
                    pltpu.make_async_copy(mbuf.at[nb], shared_x.at[ibuf.at[nb]],
                                          ssem[nb]).wait()
                if b == 1:
                    drain()
                else:
                    pl.when(p > 0)(drain)

                def prefetch():
                    loads(cid + 1, nb)
                if b == 0:
                    prefetch()
                else:
                    pl.when(p < NPAIR - 1)(prefetch)

                pltpu.make_async_copy(msg_hbm.at[pl.ds(cid * CH, CH)], mbuf.at[b],
                                      lsem[b]).wait()
                pltpu.make_async_copy(idx_hbm.at[cid, 2], ibuf.at[b], lsem[b]).wait()
                pltpu.async_copy(mbuf.at[b], shared_x.at[ibuf.at[b]], ssem[b],
                                 add=True)

        pltpu.make_async_copy(mbuf.at[1], shared_x.at[ibuf.at[1]], ssem[1]).wait()

        plsc.subcore_barrier()
        for kk in range(nfull):
            pltpu.sync_copy(shared_x.at[pl.ds(r0 + CH * kk, CH)], mbuf.at[0])
            pltpu.sync_copy(mbuf.at[0], xacc_hbm.at[c, pl.ds(r0 + CH * kk, CH)])
        if rem:
            pltpu.sync_copy(shared_x.at[pl.ds(r0 + CH * nfull, rem)],
                            mbuf.at[0, pl.ds(0, rem)])
            pltpu.sync_copy(mbuf.at[0, pl.ds(0, rem)],
                            xacc_hbm.at[c, pl.ds(r0 + CH * nfull, rem)])

    return k(msg, idxcat)


def _combine(xacc, pacc, N):
    _, NP, D = xacc.shape
    L = pacc.shape[2]
    BN = 1000
    assert N % BN == 0

    def body(x_ref, p_ref, ax_ref, ap_ref):
        ax_ref[...] = x_ref[0] + x_ref[1]
        ps = p_ref[0] + p_ref[1]
        ap_ref[...] = ps[:, :3]

    return pl.pallas_call(
        body,
        grid=(N // BN,),
        in_specs=[
            pl.BlockSpec((NC, BN, D), lambda i: (0, i, 0)),
            pl.BlockSpec((NC, BN, L), lambda i: (0, i, 0)),
        ],
        out_specs=[
            pl.BlockSpec((BN, D), lambda i: (i, 0)),
            pl.BlockSpec((BN, 3), lambda i: (i, 0)),
        ],
        out_shape=[
            jax.ShapeDtypeStruct((N, D), jnp.float32),
            jax.ShapeDtypeStruct((N, 3), jnp.float32),
        ],
    )(xacc, pacc)


def kernel(x, pos, edge_index, W1, b1, W2, b2, W3, b3, W4, b4):
    N, D = x.shape
    E = edge_index.shape[1]
    H = W1.shape[0]

    PWC = -(-E // (NW * CH))
    PWC = -(-PWC // 4) * 4
    E_pad = NW * CH * PWC
    PAD = E_pad - E
    NDUM = 64
    NP = -(-(N + NDUM) // (NS * 8)) * (NS * 8)

    W1aT = W1[:, :D].T
    W1bT = W1[:, D:2 * D].T
    w1c = W1[:, 2 * D]
    W2T = W2.T
    W3T = W3.T
    b1r = b1.reshape(1, H)
    b2r = b2.reshape(1, D)
    b3r = b3.reshape(1, H)
    w4row = W4.reshape(1, H)
    b4r = b4.reshape(1, 1)

    xa, xbb, wp = _node_tables(x, W1aT, W1bT, b1r, W3T, b3r, w4row, b4r)

    zpadT = jnp.zeros((N, 12), jnp.float32)
    zpadU = jnp.zeros((N, 13), jnp.float32)
    T = jnp.concatenate([xa, pos, wp, zpadT], axis=1)
    U = jnp.concatenate([xbb, pos, zpadU], axis=1)

    row = edge_index[0]
    col = edge_index[1]
    rowp = jnp.concatenate([row, jnp.zeros((PAD,), jnp.int32)])
    colg = jnp.concatenate([col, jnp.zeros((PAD,), jnp.int32)])
    cols = jnp.concatenate(
        [col, (N + jnp.arange(PAD, dtype=jnp.int32) % NDUM)])
    idxcat = (jnp.stack([rowp, colg, cols], axis=0)
              .reshape(3, E_pad // CH, CH)
              .transpose(1, 0, 2))

    g, pacc = _edge_gather_g(T, U, idxcat, w1c, E_pad, NP, H)
    msg = _edge_mlp(g, W2T, b2r)
    xacc = _scatter_msg(msg, idxcat, NP)
    aggregated_x, aggregated_pos = _combine(xacc, pacc, N)
    return (aggregated_x, aggregated_pos)


# speedup vs baseline: 4.1831x; 1.0263x over previous
import functools

import jax
import jax.numpy as jnp
from jax import lax
from jax.experimental import pallas as pl
from jax.experimental.pallas import tpu as pltpu
from jax.experimental.pallas import tpu_sc as plsc

NC = 2
NS = 16
NW = NC * NS
LANES = 16
CH = 128


def _node_tables(x, W1aT, W1bT, b1, W3T, b3, w4row, b4):
    N, D = x.shape
    H = W1aT.shape[1]
    BN = 1000
    assert N % BN == 0

    def body(x_ref, w1a_ref, w1b_ref, b1_ref, w3_ref, b3_ref, w4_ref, b4_ref,
             xa_ref, xbb_ref, wp_ref):
        xb = x_ref[...]
        xa_ref[...] = jnp.dot(xb, w1a_ref[...], preferred_element_type=jnp.float32)
        xbb_ref[...] = jnp.dot(xb, w1b_ref[...], preferred_element_type=jnp.float32) + b1_ref[...]
        h2 = jax.nn.silu(jnp.dot(xb, w3_ref[...], preferred_element_type=jnp.float32) + b3_ref[...])
        wp_ref[...] = jnp.sum(h2 * w4_ref[...], axis=1, keepdims=True) + b4_ref[...]

    return pl.pallas_call(
        body,
        grid=(N // BN,),
        in_specs=[
            pl.BlockSpec((BN, D), lambda i: (i, 0)),
            pl.BlockSpec((D, H), lambda i: (0, 0)),
            pl.BlockSpec((D, H), lambda i: (0, 0)),
            pl.BlockSpec((1, H), lambda i: (0, 0)),
            pl.BlockSpec((D, H), lambda i: (0, 0)),
            pl.BlockSpec((1, H), lambda i: (0, 0)),
            pl.BlockSpec((1, H), lambda i: (0, 0)),
            pl.BlockSpec((1, 1), lambda i: (0, 0)),
        ],
        out_specs=[
            pl.BlockSpec((BN, H), lambda i: (i, 0)),
            pl.BlockSpec((BN, H), lambda i: (i, 0)),
            pl.BlockSpec((BN, 1), lambda i: (i, 0)),
        ],
        out_shape=[
            jax.ShapeDtypeStruct((N, H), jnp.float32),
            jax.ShapeDtypeStruct((N, H), jnp.float32),
            jax.ShapeDtypeStruct((N, 1), jnp.float32),
        ],
    )(x, W1aT, W1bT, b1, W3T, b3, w4row, b4)


def _edge_gather_g(T, U, idxcat, w1c, E_pad, NP, H):
    TW = T.shape[1]
    PWC = E_pad // (NW * CH)
    NPAIR = PWC // 2
    RPS = NP // NS
    NJ = H // LANES
    nfull, rem = RPS // CH, RPS % CH
    mesh = plsc.VectorSubcoreMesh(core_axis_name="c", subcore_axis_name="s")

    @functools.partial(
        pl.kernel,
        mesh=mesh,
        compiler_params=pltpu.CompilerParams(use_tc_tiling_on_sc=False),
        out_type=[
            jax.ShapeDtypeStruct((E_pad, H), jnp.float32),
            jax.ShapeDtypeStruct((NC, NP, LANES), jnp.float32),
        ],
        scratch_types=[
            pltpu.VMEM((2, 3, CH), jnp.int32),
            pltpu.VMEM((2, CH), jnp.int32),
            pltpu.VMEM((2, CH, TW), jnp.float32),
            pltpu.VMEM((2, CH, TW), jnp.float32),
            pltpu.VMEM((2, CH, H), jnp.float32),
            pltpu.VMEM((2, CH, LANES), jnp.float32),
            pltpu.VMEM((H,), jnp.float32),
            pltpu.VMEM_SHARED((NP, LANES), jnp.float32),
            pltpu.SemaphoreType.DMA,
            pltpu.SemaphoreType.DMA,
            pltpu.SemaphoreType.DMA,
            pltpu.SemaphoreType.DMA,
            pltpu.SemaphoreType.DMA,
            pltpu.SemaphoreType.DMA,
            pltpu.SemaphoreType.DMA,
            pltpu.SemaphoreType.DMA,
        ],
    )
    def k(t_hbm, u_hbm, idx_hbm, w1c_hbm,
          g_hbm, pacc_hbm,
          ibuf, sbuf, tbuf, ubuf, gbuf, pubuf, w1cv, shared_pos,
          isem0, isem1, gsm0, gsm1, stm0, stm1, scm0, scm1):
        c = lax.axis_index("c")
        s = lax.axis_index("s")
        wid = s * NC + c
        isem = (isem0, isem1)
        gsm = (gsm0, gsm1)
        stm = (stm0, stm1)
        scm = (scm0, scm1)

        pltpu.sync_copy(w1c_hbm, w1cv)
        w1cs = [w1cv[pl.ds(LANES * j, LANES)] for j in range(NJ)]
        io = lax.iota(jnp.int32, LANES)
        mask3 = jnp.where(io < 3, 1.0, 0.0).astype(jnp.float32)
        zero16 = jnp.zeros((LANES,), jnp.float32)

        @pl.loop(0, CH)
        def _(r):
            pubuf[0, r, :] = zero16

        r0 = s * RPS
        for kk in range(nfull):
            pltpu.sync_copy(pubuf.at[0], shared_pos.at[pl.ds(r0 + CH * kk, CH)])
        if rem:
            pltpu.sync_copy(pubuf.at[0, pl.ds(0, rem)],
                            shared_pos.at[pl.ds(r0 + CH * nfull, rem)])
        plsc.subcore_barrier()

        cid0 = wid * PWC

        def gathers(cid, b):
            cp1 = pltpu.async_copy(t_hbm.at[ibuf.at[b, 0]], tbuf.at[b], gsm[b])
            cp2 = pltpu.async_copy(u_hbm.at[ibuf.at[b, 1]], ubuf.at[b], gsm[b])
            return cp1, cp2

        pltpu.sync_copy(idx_hbm.at[cid0], ibuf.at[0])
        gathers(cid0, 0)
        pltpu.async_copy(idx_hbm.at[cid0 + 1], ibuf.at[1], isem1)

        @pl.loop(0, NPAIR)
        def _(p):
            for b in (0, 1):
                nb = 1 - b
                ch = 2 * p + b
                cid = cid0 + ch
                e0 = cid * CH

                pltpu.make_async_copy(t_hbm.at[ibuf.at[b, 0]], tbuf.at[b], gsm[b]).wait()
                pltpu.make_async_copy(u_hbm.at[ibuf.at[b, 1]], ubuf.at[b], gsm[b]).wait()

                @pl.when(p > 0)
                def _():
                    pltpu.make_async_copy(gbuf.at[b], g_hbm.at[pl.ds(e0, CH)], stm[b]).wait()
                    pltpu.make_async_copy(pubuf.at[b], shared_pos.at[sbuf.at[b]], scm[b]).wait()

                for j in range(CH // LANES):
                    sl = pl.ds(LANES * j, LANES)
                    sbuf[b, sl] = ibuf[b, 2, sl]

                @pl.when(p < NPAIR - 1)
                def _():
                    pltpu.async_copy(idx_hbm.at[cid + 2], ibuf.at[b], isem[b])

                def launch_next():
                    pltpu.make_async_copy(idx_hbm.at[cid], ibuf.at[nb], isem[nb]).wait()
                    gathers(cid + 1, nb)
                if b == 0:
                    launch_next()
                else:
                    pl.when(p < NPAIR - 1)(launch_next)

                tb = tbuf.at[b]
                ub = ubuf.at[b]
                gb = gbuf.at[b]
                pb = pubuf.at[b]

                @plsc.parallel_loop(0, CH, unroll=4)
                def _(e):
                    t8 = tb[e, pl.ds(H, LANES)]
                    u8 = ub[e, pl.ds(H, LANES)]
                    r = t8 - u8
                    rel = r * mask3
                    d = r[0] * r[0] + r[1] * r[1] + r[2] * r[2]
                    wp = t8[3]
                    pb[e, :] = wp * rel
                    for j in range(NJ):
                        sl = pl.ds(LANES * j, LANES)
                        gb[e, sl] = tb[e, sl] + ub[e, sl] + d * w1cs[j]

                pltpu.async_copy(gbuf.at[b], g_hbm.at[pl.ds(e0, CH)], stm[b])
                pltpu.async_copy(pubuf.at[b], shared_pos.at[sbuf.at[b]], scm[b],
                                 add=True)

        for b in (0, 1):
            pltpu.make_async_copy(gbuf.at[b], g_hbm.at[pl.ds(0, CH)], stm[b]).wait()
            pltpu.make_async_copy(pubuf.at[b], shared_pos.at[sbuf.at[b]], scm[b]).wait()

        plsc.subcore_barrier()
        for kk in range(nfull):
            pltpu.sync_copy(shared_pos.at[pl.ds(r0 + CH * kk, CH)], pubuf.at[0])
            pltpu.sync_copy(pubuf.at[0], pacc_hbm.at[c, pl.ds(r0 + CH * kk, CH)])
        if rem:
            pltpu.sync_copy(shared_pos.at[pl.ds(r0 + CH * nfull, rem)],
                            pubuf.at[0, pl.ds(0, rem)])
            pltpu.sync_copy(pubuf.at[0, pl.ds(0, rem)],
                            pacc_hbm.at[c, pl.ds(r0 + CH * nfull, rem)])

    return k(T, U, idxcat, w1c)


def _edge_mlp(g, W2T, b2):
    E_pad, H = g.shape
    D = W2T.shape[1]
    BE = 1024
    assert E_pad % BE == 0

    def body(g_ref, w2_ref, b2_ref, msg_ref):
        h = jax.nn.silu(g_ref[...])
        msg_ref[...] = jnp.dot(h, w2_ref[...], preferred_element_type=jnp.float32) + b2_ref[...]

    return pl.pallas_call(
        body,
        grid=(E_pad // BE,),
        in_specs=[
            pl.BlockSpec((BE, H), lambda i: (i, 0)),
            pl.BlockSpec((H, D), lambda i: (0, 0)),
            pl.BlockSpec((1, D), lambda i: (0, 0)),
        ],
        out_specs=pl.BlockSpec((BE, D), lambda i: (i, 0)),
        out_shape=jax.ShapeDtypeStruct((E_pad, D), jnp.float32),
    )(g, W2T, b2)


def _scatter_msg(msg, idxcat, NP):
    E_pad, D = msg.shape
    PWC = E_pad // (NW * CH)
    assert PWC % 2 == 0
    NPAIR = PWC // 2
    RPS = NP // NS
    nfull, rem = RPS // CH, RPS % CH
    mesh = plsc.VectorSubcoreMesh(core_axis_name="c", subcore_axis_name="s")

    @functools.partial(
        pl.kernel,
        mesh=mesh,
        compiler_params=pltpu.CompilerParams(use_tc_tiling_on_sc=False),
        out_type=jax.ShapeDtypeStruct((NC, NP, D), jnp.float32),
        scratch_types=[
            pltpu.VMEM((2, CH), jnp.int32),
            pltpu.VMEM((2, CH, D), jnp.float32),
            pltpu.VMEM_SHARED((NP, D), jnp.float32),
            pltpu.SemaphoreType.DMA,
            pltpu.SemaphoreType.DMA,
            pltpu.SemaphoreType.DMA,
            pltpu.SemaphoreType.DMA,
        ],
    )
    def k(msg_hbm, idx_hbm, xacc_hbm, ibuf, mbuf, shared_x,
          lsem0, lsem1, ssem0, ssem1):
        c = lax.axis_index("c")
        s = lax.axis_index("s")
        wid = s * NC + c
        lsem = (lsem0, lsem1)
        ssem = (ssem0, ssem1)
        zero16 = jnp.zeros((LANES,), jnp.float32)

        @pl.loop(0, CH)
        def _(r):
            for j in range(D // LANES):
                mbuf[0, r, pl.ds(LANES * j, LANES)] = zero16

        r0 = s * RPS
        for kk in range(nfull):
            pltpu.sync_copy(mbuf.at[0], shared_x.at[pl.ds(r0 + CH * kk, CH)])
        if rem:
            pltpu.sync_copy(mbuf.at[0, pl.ds(0, rem)],
                            shared_x.at[pl.ds(r0 + CH * nfull, rem)])
        plsc.subcore_barrier()

        cid0 = wid * PWC

        def loads(cid, b):
            pltpu.async_copy(msg_hbm.at[pl.ds(cid * CH, CH)], mbuf.at[b], lsem[b])
            pltpu.async_copy(idx_hbm.at[cid, 2], ibuf.at[b], lsem[b])

        loads(cid0, 0)

        @pl.loop(0, NPAIR)
        def _(p):
            for b in (0, 1):
                nb = 1 - b
                ch = 2 * p + b
                cid = cid0 + ch

                def drain():
                    pltpu.make_async_copy(mbuf.at[nb], shared_x.at[ibuf.at[nb]],
                                          ssem[nb]).wait()
                if b == 1:
                    drain()
                else:
                    pl.when(p > 0)(drain)

                def prefetch():
                    loads(cid + 1, nb)
                if b == 0:
                    prefetch()
                else:
                    pl.when(p < NPAIR - 1)(prefetch)

                pltpu.make_async_copy(msg_hbm.at[pl.ds(cid * CH, CH)], mbuf.at[b],
                                      lsem[b]).wait()
                pltpu.make_async_copy(idx_hbm.at[cid, 2], ibuf.at[b], lsem[b]).wait()
                pltpu.async_copy(mbuf.at[b], shared_x.at[ibuf.at[b]], ssem[b],
                                 add=True)

        pltpu.make_async_copy(mbuf.at[1], shared_x.at[ibuf.at[1]], ssem[1]).wait()

        plsc.subcore_barrier()
        for kk in range(nfull):
            pltpu.sync_copy(shared_x.at[pl.ds(r0 + CH * kk, CH)], mbuf.at[0])
            pltpu.sync_copy(mbuf.at[0], xacc_hbm.at[c, pl.ds(r0 + CH * kk, CH)])
        if rem:
            pltpu.sync_copy(shared_x.at[pl.ds(r0 + CH * nfull, rem)],
                            mbuf.at[0, pl.ds(0, rem)])
            pltpu.sync_copy(mbuf.at[0, pl.ds(0, rem)],
                            xacc_hbm.at[c, pl.ds(r0 + CH * nfull, rem)])

    return k(msg, idxcat)


def _combine(xacc, pacc, N):
    _, NP, D = xacc.shape
    L = pacc.shape[2]
    BN = 1000
    assert N % BN == 0

    def body(x_ref, p_ref, ax_ref, ap_ref):
        ax_ref[...] = x_ref[0] + x_ref[1]
        ps = p_ref[0] + p_ref[1]
        ap_ref[...] = ps[:, :3]

    return pl.pallas_call(
        body,
        grid=(N // BN,),
        in_specs=[
            pl.BlockSpec((NC, BN, D), lambda i: (0, i, 0)),
            pl.BlockSpec((NC, BN, L), lambda i: (0, i, 0)),
        ],
        out_specs=[
            pl.BlockSpec((BN, D), lambda i: (i, 0)),
            pl.BlockSpec((BN, 3), lambda i: (i, 0)),
        ],
        out_shape=[
            jax.ShapeDtypeStruct((N, D), jnp.float32),
            jax.ShapeDtypeStruct((N, 3), jnp.float32),
        ],
    )(xacc, pacc)


def kernel(x, pos, edge_index, W1, b1, W2, b2, W3, b3, W4, b4):
    N, D = x.shape
    E = edge_index.shape[1]
    H = W1.shape[0]

    PWC = -(-E // (NW * CH))
    PWC = -(-PWC // 4) * 4
    E_pad = NW * CH * PWC
    PAD = E_pad - E
    NDUM = 64
    NP = -(-(N + NDUM) // (NS * 8)) * (NS * 8)

    W1aT = W1[:, :D].T
    W1bT = W1[:, D:2 * D].T
    w1c = W1[:, 2 * D]
    W2T = W2.T
    W3T = W3.T
    b1r = b1.reshape(1, H)
    b2r = b2.reshape(1, D)
    b3r = b3.reshape(1, H)
    w4row = W4.reshape(1, H)
    b4r = b4.reshape(1, 1)

    xa, xbb, wp = _node_tables(x, W1aT, W1bT, b1r, W3T, b3r, w4row, b4r)

    zpadT = jnp.zeros((N, 12), jnp.float32)
    zpadU = jnp.zeros((N, 13), jnp.float32)
    T = jnp.concatenate([xa, pos, wp, zpadT], axis=1)
    U = jnp.concatenate([xbb, pos, zpadU], axis=1)

    row = edge_index[0]
    col = edge_index[1]
    rowp = jnp.concatenate([row, jnp.zeros((PAD,), jnp.int32)])
    colg = jnp.concatenate([col, jnp.zeros((PAD,), jnp.int32)])
    cols = jnp.concatenate(
        [col, (N + jnp.arange(PAD, dtype=jnp.int32) % NDUM)])
    idxcat = (jnp.stack([rowp, colg, cols], axis=0)
              .reshape(3, E_pad // CH, CH)
              .transpose(1, 0, 2))

    g, pacc = _edge_gather_g(T, U, idxcat, w1c, E_pad, NP, H)
    msg = _edge_mlp(g, W2T, b2r)
    xacc = _scatter_msg(msg, idxcat, NP)
    aggregated_x, aggregated_pos = _combine(xacc, pacc, N)
    return (aggregated_x, aggregated_pos)


# speedup vs baseline: 4.6735x; 1.1172x over previous
import functools

import jax
import jax.numpy as jnp
from jax import lax
from jax.experimental import pallas as pl
from jax.experimental.pallas import tpu as pltpu
from jax.experimental.pallas import tpu_sc as plsc

NC = 2
NS = 16
NW = NC * NS
LANES = 16
CH = 64


def _node_tables(x, W1aT, W1bT, b1, W3T, b3, w4row, b4):
    N, D = x.shape
    H = W1aT.shape[1]
    BN = 1000
    assert N % BN == 0

    def body(x_ref, w1a_ref, w1b_ref, b1_ref, w3_ref, b3_ref, w4_ref, b4_ref,
             xa_ref, xbb_ref, wp_ref):
        xb = x_ref[...]
        xa_ref[...] = jnp.dot(xb, w1a_ref[...], preferred_element_type=jnp.float32)
        xbb_ref[...] = jnp.dot(xb, w1b_ref[...], preferred_element_type=jnp.float32) + b1_ref[...]
        h2 = jax.nn.silu(jnp.dot(xb, w3_ref[...], preferred_element_type=jnp.float32) + b3_ref[...])
        wp_ref[...] = jnp.sum(h2 * w4_ref[...], axis=1, keepdims=True) + b4_ref[...]

    return pl.pallas_call(
        body,
        grid=(N // BN,),
        in_specs=[
            pl.BlockSpec((BN, D), lambda i: (i, 0)),
            pl.BlockSpec((D, H), lambda i: (0, 0)),
            pl.BlockSpec((D, H), lambda i: (0, 0)),
            pl.BlockSpec((1, H), lambda i: (0, 0)),
            pl.BlockSpec((D, H), lambda i: (0, 0)),
            pl.BlockSpec((1, H), lambda i: (0, 0)),
            pl.BlockSpec((1, H), lambda i: (0, 0)),
            pl.BlockSpec((1, 1), lambda i: (0, 0)),
        ],
        out_specs=[
            pl.BlockSpec((BN, H), lambda i: (i, 0)),
            pl.BlockSpec((BN, H), lambda i: (i, 0)),
            pl.BlockSpec((BN, 1), lambda i: (i, 0)),
        ],
        out_shape=[
            jax.ShapeDtypeStruct((N, H), jnp.float32),
            jax.ShapeDtypeStruct((N, H), jnp.float32),
            jax.ShapeDtypeStruct((N, 1), jnp.float32),
        ],
    )(x, W1aT, W1bT, b1, W3T, b3, w4row, b4)


def _edge_gather_g(T, U, idxcat, w1c, E_pad, NP, H):
    TW = T.shape[1]
    PWC = E_pad // (NW * CH)
    NQ = PWC // 4
    RPS = NP // NS
    NJ = H // LANES
    nfull, rem = RPS // CH, RPS % CH
    mesh = plsc.VectorSubcoreMesh(core_axis_name="c", subcore_axis_name="s")

    @functools.partial(
        pl.kernel,
        mesh=mesh,
        compiler_params=pltpu.CompilerParams(use_tc_tiling_on_sc=False),
        out_type=[
            jax.ShapeDtypeStruct((E_pad, H), jnp.float32),
            jax.ShapeDtypeStruct((NC, NP, LANES), jnp.float32),
        ],
        scratch_types=[
            pltpu.VMEM((4, 3, CH), jnp.int32),
            pltpu.VMEM((2, CH), jnp.int32),
            pltpu.VMEM((4, CH, TW), jnp.float32),
            pltpu.VMEM((4, CH, TW), jnp.float32),
            pltpu.VMEM((2, CH, H), jnp.float32),
            pltpu.VMEM((2, CH, LANES), jnp.float32),
            pltpu.VMEM((H,), jnp.float32),
            pltpu.VMEM_SHARED((NP, LANES), jnp.float32),
            pltpu.SemaphoreType.DMA,
            pltpu.SemaphoreType.DMA,
            pltpu.SemaphoreType.DMA,
            pltpu.SemaphoreType.DMA,
            pltpu.SemaphoreType.DMA,
            pltpu.SemaphoreType.DMA,
            pltpu.SemaphoreType.DMA,
            pltpu.SemaphoreType.DMA,
            pltpu.SemaphoreType.DMA,
            pltpu.SemaphoreType.DMA,
            pltpu.SemaphoreType.DMA,
            pltpu.SemaphoreType.DMA,
        ],
    )
    def k(t_hbm, u_hbm, idx_hbm, w1c_hbm,
          g_hbm, pacc_hbm,
          ibuf, sbuf, tbuf, ubuf, gbuf, pubuf, w1cv, shared_pos,
          isem0, isem1, isem2, isem3, gsm0, gsm1, gsm2, gsm3,
          stm0, stm1, scm0, scm1):
        c = lax.axis_index("c")
        s = lax.axis_index("s")
        wid = s * NC + c
        isem = (isem0, isem1, isem2, isem3)
        gsm = (gsm0, gsm1, gsm2, gsm3)
        stm = (stm0, stm1)
        scm = (scm0, scm1)

        pltpu.sync_copy(w1c_hbm, w1cv)
        w1cs = [w1cv[pl.ds(LANES * j, LANES)] for j in range(NJ)]
        io = lax.iota(jnp.int32, LANES)
        mask3 = jnp.where(io < 3, 1.0, 0.0).astype(jnp.float32)
        zero16 = jnp.zeros((LANES,), jnp.float32)

        @pl.loop(0, CH)
        def _(r):
            pubuf[0, r, :] = zero16

        r0 = s * RPS
        for kk in range(nfull):
            pltpu.sync_copy(pubuf.at[0], shared_pos.at[pl.ds(r0 + CH * kk, CH)])
        if rem:
            pltpu.sync_copy(pubuf.at[0, pl.ds(0, rem)],
                            shared_pos.at[pl.ds(r0 + CH * nfull, rem)])
        plsc.subcore_barrier()

        cid0 = wid * PWC

        def gathers(b):
            pltpu.async_copy(t_hbm.at[ibuf.at[b, 0]], tbuf.at[b], gsm[b])
            pltpu.async_copy(u_hbm.at[ibuf.at[b, 1]], ubuf.at[b], gsm[b])

        pltpu.sync_copy(idx_hbm.at[cid0], ibuf.at[0])
        pltpu.sync_copy(idx_hbm.at[cid0 + 1], ibuf.at[1])
        gathers(0)
        gathers(1)
        pltpu.async_copy(idx_hbm.at[cid0 + 2], ibuf.at[2], isem[2])
        pltpu.async_copy(idx_hbm.at[cid0 + 3], ibuf.at[3], isem[3])

        @pl.loop(0, NQ)
        def _(p):
            for b in range(4):
                q = b % 2
                b2 = (b + 2) % 4
                ch = 4 * p + b
                cid = cid0 + ch
                e0 = cid * CH

                pltpu.make_async_copy(t_hbm.at[ibuf.at[b, 0]], tbuf.at[b], gsm[b]).wait()
                pltpu.make_async_copy(u_hbm.at[ibuf.at[b, 1]], ubuf.at[b], gsm[b]).wait()

                def drain():
                    pltpu.make_async_copy(gbuf.at[q], g_hbm.at[pl.ds(e0, CH)], stm[q]).wait()
                    pltpu.make_async_copy(pubuf.at[q], shared_pos.at[sbuf.at[q]], scm[q]).wait()
                if b < 2:
                    pl.when(p > 0)(drain)
                else:
                    drain()

                for j in range(CH // LANES):
                    sl = pl.ds(LANES * j, LANES)
                    sbuf[q, sl] = ibuf[b, 2, sl]

                @pl.when(p < NQ - 1)
                def _():
                    pltpu.async_copy(idx_hbm.at[cid + 4], ibuf.at[b], isem[b])

                def launch_next():
                    pltpu.make_async_copy(idx_hbm.at[cid], ibuf.at[b2], isem[b2]).wait()
                    gathers(b2)
                if b < 2:
                    launch_next()
                else:
                    pl.when(p < NQ - 1)(launch_next)

                tb = tbuf.at[b]
                ub = ubuf.at[b]
                gb = gbuf.at[q]
                pb = pubuf.at[q]

                @plsc.parallel_loop(0, CH, unroll=4)
                def _(e):
                    t8 = tb[e, pl.ds(H, LANES)]
                    u8 = ub[e, pl.ds(H, LANES)]
                    r = t8 - u8
                    rel = r * mask3
                    d = r[0] * r[0] + r[1] * r[1] + r[2] * r[2]
                    wp = t8[3]
                    pb[e, :] = wp * rel
                    for j in range(NJ):
                        sl = pl.ds(LANES * j, LANES)
                        gb[e, sl] = tb[e, sl] + ub[e, sl] + d * w1cs[j]

                pltpu.async_copy(gbuf.at[q], g_hbm.at[pl.ds(e0, CH)], stm[q])
                pltpu.async_copy(pubuf.at[q], shared_pos.at[sbuf.at[q]], scm[q],
                                 add=True)

        for q in (0, 1):
            pltpu.make_async_copy(gbuf.at[q], g_hbm.at[pl.ds(0, CH)], stm[q]).wait()
            pltpu.make_async_copy(pubuf.at[q], shared_pos.at[sbuf.at[q]], scm[q]).wait()

        plsc.subcore_barrier()
        for kk in range(nfull):
            pltpu.sync_copy(shared_pos.at[pl.ds(r0 + CH * kk, CH)], pubuf.at[0])
            pltpu.sync_copy(pubuf.at[0], pacc_hbm.at[c, pl.ds(r0 + CH * kk, CH)])
        if rem:
            pltpu.sync_copy(shared_pos.at[pl.ds(r0 + CH * nfull, rem)],
                            pubuf.at[0, pl.ds(0, rem)])
            pltpu.sync_copy(pubuf.at[0, pl.ds(0, rem)],
                            pacc_hbm.at[c, pl.ds(r0 + CH * nfull, rem)])

    return k(T, U, idxcat, w1c)


def _edge_mlp(g, W2T, b2):
    E_pad, H = g.shape
    D = W2T.shape[1]
    BE = 2048
    assert E_pad % BE == 0

    def body(g_ref, w2_ref, b2_ref, msg_ref):
        h = jax.nn.silu(g_ref[...]).astype(jnp.bfloat16)
        msg_ref[...] = jnp.dot(h, w2_ref[...], preferred_element_type=jnp.float32) + b2_ref[...]

    return pl.pallas_call(
        body,
        grid=(E_pad // BE,),
        in_specs=[
            pl.BlockSpec((BE, H), lambda i: (i, 0)),
            pl.BlockSpec((H, D), lambda i: (0, 0)),
            pl.BlockSpec((1, D), lambda i: (0, 0)),
        ],
        out_specs=pl.BlockSpec((BE, D), lambda i: (i, 0)),
        out_shape=jax.ShapeDtypeStruct((E_pad, D), jnp.float32),
    )(g, W2T, b2)


def _scatter_msg(msg, idxcat, NP):
    E_pad, D = msg.shape
    PWC = E_pad // (NW * CH)
    assert PWC % 2 == 0
    NPAIR = PWC // 2
    RPS = NP // NS
    nfull, rem = RPS // CH, RPS % CH
    mesh = plsc.VectorSubcoreMesh(core_axis_name="c", subcore_axis_name="s")

    @functools.partial(
        pl.kernel,
        mesh=mesh,
        compiler_params=pltpu.CompilerParams(use_tc_tiling_on_sc=False),
        out_type=jax.ShapeDtypeStruct((NC, NP, D), jnp.float32),
        scratch_types=[
            pltpu.VMEM((2, CH), jnp.int32),
            pltpu.VMEM((2, CH, D), jnp.float32),
            pltpu.VMEM_SHARED((NP, D), jnp.float32),
            pltpu.SemaphoreType.DMA,
            pltpu.SemaphoreType.DMA,
            pltpu.SemaphoreType.DMA,
            pltpu.SemaphoreType.DMA,
        ],
    )
    def k(msg_hbm, idx_hbm, xacc_hbm, ibuf, mbuf, shared_x,
          lsem0, lsem1, ssem0, ssem1):
        c = lax.axis_index("c")
        s = lax.axis_index("s")
        wid = s * NC + c
        lsem = (lsem0, lsem1)
        ssem = (ssem0, ssem1)
        zero16 = jnp.zeros((LANES,), jnp.float32)

        @pl.loop(0, CH)
        def _(r):
            for j in range(D // LANES):
                mbuf[0, r, pl.ds(LANES * j, LANES)] = zero16

        r0 = s * RPS
        for kk in range(nfull):
            pltpu.sync_copy(mbuf.at[0], shared_x.at[pl.ds(r0 + CH * kk, CH)])
        if rem:
            pltpu.sync_copy(mbuf.at[0, pl.ds(0, rem)],
                            shared_x.at[pl.ds(r0 + CH * nfull, rem)])
        plsc.subcore_barrier()

        cid0 = wid * PWC

        def loads(cid, b):
            pltpu.async_copy(msg_hbm.at[pl.ds(cid * CH, CH)], mbuf.at[b], lsem[b])
            pltpu.async_copy(idx_hbm.at[cid, 2], ibuf.at[b], lsem[b])

        loads(cid0, 0)

        @pl.loop(0, NPAIR)
        def _(p):
            for b in (0, 1):
                nb = 1 - b
                ch = 2 * p + b
                cid = cid0 + ch

                def drain():
                    pltpu.make_async_copy(mbuf.at[nb], shared_x.at[ibuf.at[nb]],
                                          ssem[nb]).wait()
                if b == 1:
                    drain()
                else:
                    pl.when(p > 0)(drain)

                def prefetch():
                    loads(cid + 1, nb)
                if b == 0:
                    prefetch()
                else:
                    pl.when(p < NPAIR - 1)(prefetch)

                pltpu.make_async_copy(msg_hbm.at[pl.ds(cid * CH, CH)], mbuf.at[b],
                                      lsem[b]).wait()
                pltpu.make_async_copy(idx_hbm.at[cid, 2], ibuf.at[b], lsem[b]).wait()
                pltpu.async_copy(mbuf.at[b], shared_x.at[ibuf.at[b]], ssem[b],
                                 add=True)

        pltpu.make_async_copy(mbuf.at[1], shared_x.at[ibuf.at[1]], ssem[1]).wait()

        plsc.subcore_barrier()
        for kk in range(nfull):
            pltpu.sync_copy(shared_x.at[pl.ds(r0 + CH * kk, CH)], mbuf.at[0])
            pltpu.sync_copy(mbuf.at[0], xacc_hbm.at[c, pl.ds(r0 + CH * kk, CH)])
        if rem:
            pltpu.sync_copy(shared_x.at[pl.ds(r0 + CH * nfull, rem)],
                            mbuf.at[0, pl.ds(0, rem)])
            pltpu.sync_copy(mbuf.at[0, pl.ds(0, rem)],
                            xacc_hbm.at[c, pl.ds(r0 + CH * nfull, rem)])

    return k(msg, idxcat)


def _combine(xacc, pacc, N):
    _, NP, D = xacc.shape
    L = pacc.shape[2]
    BN = 1000
    assert N % BN == 0

    def body(x_ref, p_ref, ax_ref, ap_ref):
        ax_ref[...] = x_ref[0] + x_ref[1]
        ps = p_ref[0] + p_ref[1]
        ap_ref[...] = ps[:, :3]

    return pl.pallas_call(
        body,
        grid=(N // BN,),
        in_specs=[
            pl.BlockSpec((NC, BN, D), lambda i: (0, i, 0)),
            pl.BlockSpec((NC, BN, L), lambda i: (0, i, 0)),
        ],
        out_specs=[
            pl.BlockSpec((BN, D), lambda i: (i, 0)),
            pl.BlockSpec((BN, 3), lambda i: (i, 0)),
        ],
        out_shape=[
            jax.ShapeDtypeStruct((N, D), jnp.float32),
            jax.ShapeDtypeStruct((N, 3), jnp.float32),
        ],
    )(xacc, pacc)


def kernel(x, pos, edge_index, W1, b1, W2, b2, W3, b3, W4, b4):
    N, D = x.shape
    E = edge_index.shape[1]
    H = W1.shape[0]

    PWC = -(-E // (NW * CH))
    PWC = -(-PWC // 4) * 4
    E_pad = NW * CH * PWC
    PAD = E_pad - E
    NDUM = 64
    NP = -(-(N + NDUM) // (NS * 8)) * (NS * 8)

    W1aT = W1[:, :D].T
    W1bT = W1[:, D:2 * D].T
    w1c = W1[:, 2 * D]
    W2T = W2.T.astype(jnp.bfloat16)
    W3T = W3.T
    b1r = b1.reshape(1, H)
    b2r = b2.reshape(1, D)
    b3r = b3.reshape(1, H)
    w4row = W4.reshape(1, H)
    b4r = b4.reshape(1, 1)

    xa, xbb, wp = _node_tables(x, W1aT, W1bT, b1r, W3T, b3r, w4row, b4r)

    zpadT = jnp.zeros((N, 12), jnp.float32)
    zpadU = jnp.zeros((N, 13), jnp.float32)
    T = jnp.concatenate([xa, pos, wp, zpadT], axis=1)
    U = jnp.concatenate([xbb, pos, zpadU], axis=1)

    row = edge_index[0]
    col = edge_index[1]
    rowp = jnp.concatenate([row, jnp.zeros((PAD,), jnp.int32)])
    colg = jnp.concatenate([col, jnp.zeros((PAD,), jnp.int32)])
    cols = jnp.concatenate(
        [col, (N + jnp.arange(PAD, dtype=jnp.int32) % NDUM)])
    idxcat = (jnp.stack([rowp, colg, cols], axis=0)
              .reshape(3, E_pad // CH, CH)
              .transpose(1, 0, 2))

    g, pacc = _edge_gather_g(T, U, idxcat, w1c, E_pad, NP, H)
    msg = _edge_mlp(g, W2T, b2r)
    xacc = _scatter_msg(msg, idxcat, NP)
    aggregated_x, aggregated_pos = _combine(xacc, pacc, N)
    return (aggregated_x, aggregated_pos)


# speedup vs baseline: 7.6344x; 1.6335x over previous
import functools

import jax
import jax.numpy as jnp
from jax import lax
from jax.experimental import pallas as pl
from jax.experimental.pallas import tpu as pltpu
from jax.experimental.pallas import tpu_sc as plsc

NC = 2
NS = 16
NW = NC * NS
LANES = 16
CH = 64


def _node_tables(x, W1aT, W1bT, b1, W3T, b3, w4row, b4):
    N, D = x.shape
    H = W1aT.shape[1]
    BN = 1000
    assert N % BN == 0

    def body(x_ref, w1a_ref, w1b_ref, b1_ref, w3_ref, b3_ref, w4_ref, b4_ref,
             xa_ref, xbb_ref, wp_ref):
        xb = x_ref[...]
        xa_ref[...] = jnp.dot(xb, w1a_ref[...], preferred_element_type=jnp.float32)
        xbb_ref[...] = jnp.dot(xb, w1b_ref[...], preferred_element_type=jnp.float32) + b1_ref[...]
        h2 = jax.nn.silu(jnp.dot(xb, w3_ref[...], preferred_element_type=jnp.float32) + b3_ref[...])
        wp_ref[...] = jnp.sum(h2 * w4_ref[...], axis=1, keepdims=True) + b4_ref[...]

    return pl.pallas_call(
        body,
        grid=(N // BN,),
        in_specs=[
            pl.BlockSpec((BN, D), lambda i: (i, 0)),
            pl.BlockSpec((D, H), lambda i: (0, 0)),
            pl.BlockSpec((D, H), lambda i: (0, 0)),
            pl.BlockSpec((1, H), lambda i: (0, 0)),
            pl.BlockSpec((D, H), lambda i: (0, 0)),
            pl.BlockSpec((1, H), lambda i: (0, 0)),
            pl.BlockSpec((1, H), lambda i: (0, 0)),
            pl.BlockSpec((1, 1), lambda i: (0, 0)),
        ],
        out_specs=[
            pl.BlockSpec((BN, H), lambda i: (i, 0)),
            pl.BlockSpec((BN, H), lambda i: (i, 0)),
            pl.BlockSpec((BN, 1), lambda i: (i, 0)),
        ],
        out_shape=[
            jax.ShapeDtypeStruct((N, H), jnp.float32),
            jax.ShapeDtypeStruct((N, H), jnp.float32),
            jax.ShapeDtypeStruct((N, 1), jnp.float32),
        ],
    )(x, W1aT, W1bT, b1, W3T, b3, w4row, b4)


def _edge_gather_g(T, U, idxcat, w1c, E_pad, NP, H):
    TW = T.shape[1]
    PWC = E_pad // (NW * CH)
    NQ = PWC // 4
    RPS = NP // NS
    NJ = H // LANES
    nfull, rem = RPS // CH, RPS % CH
    mesh = plsc.VectorSubcoreMesh(core_axis_name="c", subcore_axis_name="s")

    @functools.partial(
        pl.kernel,
        mesh=mesh,
        compiler_params=pltpu.CompilerParams(use_tc_tiling_on_sc=False),
        out_type=[
            jax.ShapeDtypeStruct((E_pad, H), jnp.float32),
            jax.ShapeDtypeStruct((NC, NP, LANES), jnp.float32),
        ],
        scratch_types=[
            pltpu.VMEM((4, 3, CH), jnp.int32),
            pltpu.VMEM((2, CH), jnp.int32),
            pltpu.VMEM((4, CH, TW), jnp.float32),
            pltpu.VMEM((4, CH, TW), jnp.float32),
            pltpu.VMEM((2, CH, H), jnp.float32),
            pltpu.VMEM((2, CH, LANES), jnp.float32),
            pltpu.VMEM((H,), jnp.float32),
            pltpu.VMEM_SHARED((NP, LANES), jnp.float32),
            pltpu.SemaphoreType.DMA,
            pltpu.SemaphoreType.DMA,
            pltpu.SemaphoreType.DMA,
            pltpu.SemaphoreType.DMA,
            pltpu.SemaphoreType.DMA,
            pltpu.SemaphoreType.DMA,
            pltpu.SemaphoreType.DMA,
            pltpu.SemaphoreType.DMA,
            pltpu.SemaphoreType.DMA,
            pltpu.SemaphoreType.DMA,
            pltpu.SemaphoreType.DMA,
            pltpu.SemaphoreType.DMA,
        ],
    )
    def k(t_hbm, u_hbm, idx_hbm, w1c_hbm,
          g_hbm, pacc_hbm,
          ibuf, sbuf, tbuf, ubuf, gbuf, pubuf, w1cv, shared_pos,
          isem0, isem1, isem2, isem3, gsm0, gsm1, gsm2, gsm3,
          stm0, stm1, scm0, scm1):
        c = lax.axis_index("c")
        s = lax.axis_index("s")
        wid = s * NC + c
        isem = (isem0, isem1, isem2, isem3)
        gsm = (gsm0, gsm1, gsm2, gsm3)
        stm = (stm0, stm1)
        scm = (scm0, scm1)

        pltpu.sync_copy(w1c_hbm, w1cv)
        w1cs = [w1cv[pl.ds(LANES * j, LANES)] for j in range(NJ)]
        io = lax.iota(jnp.int32, LANES)
        mask3 = jnp.where(io < 3, 1.0, 0.0).astype(jnp.float32)
        zero16 = jnp.zeros((LANES,), jnp.float32)

        @pl.loop(0, CH)
        def _(r):
            pubuf[0, r, :] = zero16

        r0 = s * RPS
        for kk in range(nfull):
            pltpu.sync_copy(pubuf.at[0], shared_pos.at[pl.ds(r0 + CH * kk, CH)])
        if rem:
            pltpu.sync_copy(pubuf.at[0, pl.ds(0, rem)],
                            shared_pos.at[pl.ds(r0 + CH * nfull, rem)])
        plsc.subcore_barrier()

        cid0 = wid * PWC

        def gathers(b):
            pltpu.async_copy(t_hbm.at[ibuf.at[b, 0]], tbuf.at[b], gsm[b])
            pltpu.async_copy(u_hbm.at[ibuf.at[b, 1]], ubuf.at[b], gsm[b])

        pltpu.sync_copy(idx_hbm.at[cid0], ibuf.at[0])
        pltpu.sync_copy(idx_hbm.at[cid0 + 1], ibuf.at[1])
        gathers(0)
        gathers(1)
        pltpu.async_copy(idx_hbm.at[cid0 + 2], ibuf.at[2], isem[2])
        pltpu.async_copy(idx_hbm.at[cid0 + 3], ibuf.at[3], isem[3])

        @pl.loop(0, NQ)
        def _(p):
            for b in range(4):
                q = b % 2
                b2 = (b + 2) % 4
                ch = 4 * p + b
                cid = cid0 + ch
                e0 = cid * CH

                pltpu.make_async_copy(t_hbm.at[ibuf.at[b, 0]], tbuf.at[b], gsm[b]).wait()
                pltpu.make_async_copy(u_hbm.at[ibuf.at[b, 1]], ubuf.at[b], gsm[b]).wait()

                def drain():
                    pltpu.make_async_copy(gbuf.at[q], g_hbm.at[pl.ds(e0, CH)], stm[q]).wait()
                    pltpu.make_async_copy(pubuf.at[q], shared_pos.at[sbuf.at[q]], scm[q]).wait()
                if b < 2:
                    pl.when(p > 0)(drain)
                else:
                    drain()

                for j in range(CH // LANES):
                    sl = pl.ds(LANES * j, LANES)
                    sbuf[q, sl] = ibuf[b, 2, sl]

                @pl.when(p < NQ - 1)
                def _():
                    pltpu.async_copy(idx_hbm.at[cid + 4], ibuf.at[b], isem[b])

                def launch_next():
                    pltpu.make_async_copy(idx_hbm.at[cid], ibuf.at[b2], isem[b2]).wait()
                    gathers(b2)
                if b < 2:
                    launch_next()
                else:
                    pl.when(p < NQ - 1)(launch_next)

                tb = tbuf.at[b]
                ub = ubuf.at[b]
                gb = gbuf.at[q]
                pb = pubuf.at[q]

                @plsc.parallel_loop(0, CH, unroll=4)
                def _(e):
                    t8 = tb[e, pl.ds(H, LANES)]
                    u8 = ub[e, pl.ds(H, LANES)]
                    r = t8 - u8
                    rel = r * mask3
                    d = r[0] * r[0] + r[1] * r[1] + r[2] * r[2]
                    wp = t8[3]
                    pb[e, :] = wp * rel
                    for j in range(NJ):
                        sl = pl.ds(LANES * j, LANES)
                        gb[e, sl] = tb[e, sl] + ub[e, sl] + d * w1cs[j]

                pltpu.async_copy(gbuf.at[q], g_hbm.at[pl.ds(e0, CH)], stm[q])
                pltpu.async_copy(pubuf.at[q], shared_pos.at[sbuf.at[q]], scm[q],
                                 add=True)

        for q in (0, 1):
            pltpu.make_async_copy(gbuf.at[q], g_hbm.at[pl.ds(0, CH)], stm[q]).wait()
            pltpu.make_async_copy(pubuf.at[q], shared_pos.at[sbuf.at[q]], scm[q]).wait()

        plsc.subcore_barrier()
        for kk in range(nfull):
            pltpu.sync_copy(shared_pos.at[pl.ds(r0 + CH * kk, CH)], pubuf.at[0])
            pltpu.sync_copy(pubuf.at[0], pacc_hbm.at[c, pl.ds(r0 + CH * kk, CH)])
        if rem:
            pltpu.sync_copy(shared_pos.at[pl.ds(r0 + CH * nfull, rem)],
                            pubuf.at[0, pl.ds(0, rem)])
            pltpu.sync_copy(pubuf.at[0, pl.ds(0, rem)],
                            pacc_hbm.at[c, pl.ds(r0 + CH * nfull, rem)])

    return k(T, U, idxcat, w1c)


def _edge_mlp(g, W2T, b2):
    E_pad, H = g.shape
    D = W2T.shape[1]
    BE = 2048
    assert E_pad % BE == 0

    def body(g_ref, w2_ref, b2_ref, msg_ref):
        h = jax.nn.silu(g_ref[...]).astype(jnp.bfloat16)
        msg_ref[...] = jnp.dot(h, w2_ref[...], preferred_element_type=jnp.float32) + b2_ref[...]

    return pl.pallas_call(
        body,
        grid=(E_pad // BE,),
        in_specs=[
            pl.BlockSpec((BE, H), lambda i: (i, 0)),
            pl.BlockSpec((H, D), lambda i: (0, 0)),
            pl.BlockSpec((1, D), lambda i: (0, 0)),
        ],
        out_specs=pl.BlockSpec((BE, D), lambda i: (i, 0)),
        out_shape=jax.ShapeDtypeStruct((E_pad, D), jnp.float32),
    )(g, W2T, b2)


def _scatter_msg(msg, idxcat, NP):
    E_pad, D = msg.shape
    PWC = E_pad // (NW * CH)
    assert PWC % 2 == 0
    NPAIR = PWC // 2
    RPS = NP // NS
    nfull, rem = RPS // CH, RPS % CH
    mesh = plsc.VectorSubcoreMesh(core_axis_name="c", subcore_axis_name="s")

    @functools.partial(
        pl.kernel,
        mesh=mesh,
        compiler_params=pltpu.CompilerParams(use_tc_tiling_on_sc=False),
        out_type=jax.ShapeDtypeStruct((NC, NP, D), jnp.float32),
        scratch_types=[
            pltpu.VMEM((2, CH), jnp.int32),
            pltpu.VMEM((2, CH, D), jnp.float32),
            pltpu.VMEM_SHARED((NP, D), jnp.float32),
            pltpu.SemaphoreType.DMA,
            pltpu.SemaphoreType.DMA,
            pltpu.SemaphoreType.DMA,
            pltpu.SemaphoreType.DMA,
        ],
    )
    def k(msg_hbm, idx_hbm, xacc_hbm, ibuf, mbuf, shared_x,
          lsem0, lsem1, ssem0, ssem1):
        c = lax.axis_index("c")
        s = lax.axis_index("s")
        wid = s * NC + c
        lsem = (lsem0, lsem1)
        ssem = (ssem0, ssem1)
        zero16 = jnp.zeros((LANES,), jnp.float32)

        @pl.loop(0, CH)
        def _(r):
            for j in range(D // LANES):
                mbuf[0, r, pl.ds(LANES * j, LANES)] = zero16

        r0 = s * RPS
        for kk in range(nfull):
            pltpu.sync_copy(mbuf.at[0], shared_x.at[pl.ds(r0 + CH * kk, CH)])
        if rem:
            pltpu.sync_copy(mbuf.at[0, pl.ds(0, rem)],
                            shared_x.at[pl.ds(r0 + CH * nfull, rem)])
        plsc.subcore_barrier()

        cid0 = wid * PWC

        def loads(cid, b):
            pltpu.async_copy(msg_hbm.at[pl.ds(cid * CH, CH)], mbuf.at[b], lsem[b])
            pltpu.async_copy(idx_hbm.at[cid, 2], ibuf.at[b], lsem[b])

        loads(cid0, 0)

        @pl.loop(0, NPAIR)
        def _(p):
            for b in (0, 1):
                nb = 1 - b
                ch = 2 * p + b
                cid = cid0 + ch

                def drain():
                    pltpu.make_async_copy(mbuf.at[nb], shared_x.at[ibuf.at[nb]],
                                          ssem[nb]).wait()
                if b == 1:
                    drain()
                else:
                    pl.when(p > 0)(drain)

                def prefetch():
                    loads(cid + 1, nb)
                if b == 0:
                    prefetch()
                else:
                    pl.when(p < NPAIR - 1)(prefetch)

                pltpu.make_async_copy(msg_hbm.at[pl.ds(cid * CH, CH)], mbuf.at[b],
                                      lsem[b]).wait()
                pltpu.make_async_copy(idx_hbm.at[cid, 2], ibuf.at[b], lsem[b]).wait()
                pltpu.async_copy(mbuf.at[b], shared_x.at[ibuf.at[b]], ssem[b],
                                 add=True)

        pltpu.make_async_copy(mbuf.at[1], shared_x.at[ibuf.at[1]], ssem[1]).wait()

        plsc.subcore_barrier()
        for kk in range(nfull):
            pltpu.sync_copy(shared_x.at[pl.ds(r0 + CH * kk, CH)], mbuf.at[0])
            pltpu.sync_copy(mbuf.at[0], xacc_hbm.at[c, pl.ds(r0 + CH * kk, CH)])
        if rem:
            pltpu.sync_copy(shared_x.at[pl.ds(r0 + CH * nfull, rem)],
                            mbuf.at[0, pl.ds(0, rem)])
            pltpu.sync_copy(mbuf.at[0, pl.ds(0, rem)],
                            xacc_hbm.at[c, pl.ds(r0 + CH * nfull, rem)])

    return k(msg, idxcat)


def _combine(xacc, pacc, N):
    _, NP, D = xacc.shape
    L = pacc.shape[2]
    BN = 1000
    assert N % BN == 0

    def body(x_ref, p_ref, ax_ref, ap_ref):
        ax_ref[...] = x_ref[0] + x_ref[1]
        ps = p_ref[0] + p_ref[1]
        ap_ref[...] = ps[:, :3]

    return pl.pallas_call(
        body,
        grid=(N // BN,),
        in_specs=[
            pl.BlockSpec((NC, BN, D), lambda i: (0, i, 0)),
            pl.BlockSpec((NC, BN, L), lambda i: (0, i, 0)),
        ],
        out_specs=[
            pl.BlockSpec((BN, D), lambda i: (i, 0)),
            pl.BlockSpec((BN, 3), lambda i: (i, 0)),
        ],
        out_shape=[
            jax.ShapeDtypeStruct((N, D), jnp.float32),
            jax.ShapeDtypeStruct((N, 3), jnp.float32),
        ],
    )(xacc, pacc)


def kernel(x, pos, edge_index, W1, b1, W2, b2, W3, b3, W4, b4):
    N, D = x.shape
    E = edge_index.shape[1]
    H = W1.shape[0]

    PWC = -(-E // (NW * CH))
    PWC = -(-PWC // 4) * 4
    E_pad = NW * CH * PWC
    PAD = E_pad - E
    NDUM = 64
    NP = -(-(N + NDUM) // (NS * 8)) * (NS * 8)

    W1aT = W1[:, :D].T
    W1bT = W1[:, D:2 * D].T
    w1c = W1[:, 2 * D]
    W2T = W2.T.astype(jnp.bfloat16)
    W3T = W3.T
    b1r = b1.reshape(1, H)
    b2r = b2.reshape(1, D)
    b3r = b3.reshape(1, H)
    w4row = W4.reshape(1, H)
    b4r = b4.reshape(1, 1)

    xa, xbb, wp = _node_tables(x, W1aT, W1bT, b1r, W3T, b3r, w4row, b4r)

    zpadT = jnp.zeros((N, 12), jnp.float32)
    zpadU = jnp.zeros((N, 13), jnp.float32)
    T = jnp.concatenate([xa, pos, wp, zpadT], axis=1)
    U = jnp.concatenate([xbb, pos, zpadU], axis=1)

    row = edge_index[0]
    col = edge_index[1]
    spread = (jnp.arange(PAD, dtype=jnp.int32) * 97) % N
    rowp = jnp.concatenate([row, spread])
    colg = jnp.concatenate([col, spread])
    cols = jnp.concatenate(
        [col, (N + jnp.arange(PAD, dtype=jnp.int32) % NDUM)])
    idxcat = (jnp.stack([rowp, colg, cols], axis=0)
              .reshape(3, E_pad // CH, CH)
              .transpose(1, 0, 2))

    g, pacc = _edge_gather_g(T, U, idxcat, w1c, E_pad, NP, H)
    msg = _edge_mlp(g, W2T, b2r)
    xacc = _scatter_msg(msg, idxcat, NP)
    aggregated_x, aggregated_pos = _combine(xacc, pacc, N)
    return (aggregated_x, aggregated_pos)


# speedup vs baseline: 8.6383x; 1.1315x over previous
import functools

import jax
import jax.numpy as jnp
from jax import lax
from jax.experimental import pallas as pl
from jax.experimental.pallas import tpu as pltpu
from jax.experimental.pallas import tpu_sc as plsc

NC = 2
NS = 16
NW = NC * NS
LANES = 16
CH = 64


def _node_tables(x, W1aT, W1bT, b1, W3T, b3, w4row, b4):
    N, D = x.shape
    H = W1aT.shape[1]
    BN = 1000
    assert N % BN == 0

    def body(x_ref, w1a_ref, w1b_ref, b1_ref, w3_ref, b3_ref, w4_ref, b4_ref,
             xa_ref, xbb_ref, wp_ref):
        xb = x_ref[...]
        xa_ref[...] = jnp.dot(xb, w1a_ref[...], preferred_element_type=jnp.float32)
        xbb_ref[...] = jnp.dot(xb, w1b_ref[...], preferred_element_type=jnp.float32) + b1_ref[...]
        h2 = jax.nn.silu(jnp.dot(xb, w3_ref[...], preferred_element_type=jnp.float32) + b3_ref[...])
        wp_ref[...] = jnp.sum(h2 * w4_ref[...], axis=1, keepdims=True) + b4_ref[...]

    return pl.pallas_call(
        body,
        grid=(N // BN,),
        in_specs=[
            pl.BlockSpec((BN, D), lambda i: (i, 0)),
            pl.BlockSpec((D, H), lambda i: (0, 0)),
            pl.BlockSpec((D, H), lambda i: (0, 0)),
            pl.BlockSpec((1, H), lambda i: (0, 0)),
            pl.BlockSpec((D, H), lambda i: (0, 0)),
            pl.BlockSpec((1, H), lambda i: (0, 0)),
            pl.BlockSpec((1, H), lambda i: (0, 0)),
            pl.BlockSpec((1, 1), lambda i: (0, 0)),
        ],
        out_specs=[
            pl.BlockSpec((BN, H), lambda i: (i, 0)),
            pl.BlockSpec((BN, H), lambda i: (i, 0)),
            pl.BlockSpec((BN, 1), lambda i: (i, 0)),
        ],
        out_shape=[
            jax.ShapeDtypeStruct((N, H), jnp.float32),
            jax.ShapeDtypeStruct((N, H), jnp.float32),
            jax.ShapeDtypeStruct((N, 1), jnp.float32),
        ],
    )(x, W1aT, W1bT, b1, W3T, b3, w4row, b4)


def _edge_gather_g(T, U, idxcat, w1c, E_pad, NP, H):
    TW = T.shape[1]
    PWC = E_pad // (NW * CH)
    NQ = PWC // 4
    RPS = NP // NS
    NJ = H // LANES
    nfull, rem = RPS // CH, RPS % CH
    mesh = plsc.VectorSubcoreMesh(core_axis_name="c", subcore_axis_name="s")

    @functools.partial(
        pl.kernel,
        mesh=mesh,
        compiler_params=pltpu.CompilerParams(use_tc_tiling_on_sc=False),
        out_type=[
            jax.ShapeDtypeStruct((E_pad, H), jnp.float32),
            jax.ShapeDtypeStruct((NC, NP, LANES), jnp.float32),
        ],
        scratch_types=[
            pltpu.VMEM((4, 3, CH), jnp.int32),
            pltpu.VMEM((2, CH), jnp.int32),
            pltpu.VMEM((4, CH, TW), jnp.float32),
            pltpu.VMEM((4, CH, TW), jnp.float32),
            pltpu.VMEM((2, CH, H), jnp.float32),
            pltpu.VMEM((2, CH, LANES), jnp.float32),
            pltpu.VMEM((H,), jnp.float32),
            pltpu.VMEM_SHARED((NP, LANES), jnp.float32),
            pltpu.SemaphoreType.DMA,
            pltpu.SemaphoreType.DMA,
            pltpu.SemaphoreType.DMA,
            pltpu.SemaphoreType.DMA,
            pltpu.SemaphoreType.DMA,
            pltpu.SemaphoreType.DMA,
            pltpu.SemaphoreType.DMA,
            pltpu.SemaphoreType.DMA,
            pltpu.SemaphoreType.DMA,
            pltpu.SemaphoreType.DMA,
            pltpu.SemaphoreType.DMA,
            pltpu.SemaphoreType.DMA,
        ],
    )
    def k(t_hbm, u_hbm, idx_hbm, w1c_hbm,
          g_hbm, pacc_hbm,
          ibuf, sbuf, tbuf, ubuf, gbuf, pubuf, w1cv, shared_pos,
          isem0, isem1, isem2, isem3, gsm0, gsm1, gsm2, gsm3,
          stm0, stm1, scm0, scm1):
        c = lax.axis_index("c")
        s = lax.axis_index("s")
        wid = s * NC + c
        isem = (isem0, isem1, isem2, isem3)
        gsm = (gsm0, gsm1, gsm2, gsm3)
        stm = (stm0, stm1)
        scm = (scm0, scm1)

        pltpu.sync_copy(w1c_hbm, w1cv)
        w1cs = [w1cv[pl.ds(LANES * j, LANES)] for j in range(NJ)]
        io = lax.iota(jnp.int32, LANES)
        mask3 = jnp.where(io < 3, 1.0, 0.0).astype(jnp.float32)
        zero16 = jnp.zeros((LANES,), jnp.float32)

        @pl.loop(0, CH)
        def _(r):
            pubuf[0, r, :] = zero16

        r0 = s * RPS
        for kk in range(nfull):
            pltpu.sync_copy(pubuf.at[0], shared_pos.at[pl.ds(r0 + CH * kk, CH)])
        if rem:
            pltpu.sync_copy(pubuf.at[0, pl.ds(0, rem)],
                            shared_pos.at[pl.ds(r0 + CH * nfull, rem)])
        plsc.subcore_barrier()

        cid0 = wid * PWC

        def gathers(b):
            pltpu.async_copy(t_hbm.at[ibuf.at[b, 0]], tbuf.at[b], gsm[b])
            pltpu.async_copy(u_hbm.at[ibuf.at[b, 1]], ubuf.at[b], gsm[b])

        pltpu.sync_copy(idx_hbm.at[cid0], ibuf.at[0])
        pltpu.sync_copy(idx_hbm.at[cid0 + 1], ibuf.at[1])
        gathers(0)
        gathers(1)
        pltpu.async_copy(idx_hbm.at[cid0 + 2], ibuf.at[2], isem[2])
        pltpu.async_copy(idx_hbm.at[cid0 + 3], ibuf.at[3], isem[3])

        @pl.loop(0, NQ)
        def _(p):
            for b in range(4):
                q = b % 2
                b2 = (b + 2) % 4
                ch = 4 * p + b
                cid = cid0 + ch
                e0 = cid * CH

                pltpu.make_async_copy(t_hbm.at[ibuf.at[b, 0]], tbuf.at[b], gsm[b]).wait()
                pltpu.make_async_copy(u_hbm.at[ibuf.at[b, 1]], ubuf.at[b], gsm[b]).wait()

                def drain():
                    pltpu.make_async_copy(gbuf.at[q], g_hbm.at[pl.ds(e0, CH)], stm[q]).wait()
                    pltpu.make_async_copy(pubuf.at[q], shared_pos.at[sbuf.at[q]], scm[q]).wait()
                if b < 2:
                    pl.when(p > 0)(drain)
                else:
                    drain()

                for j in range(CH // LANES):
                    sl = pl.ds(LANES * j, LANES)
                    sbuf[q, sl] = ibuf[b, 2, sl]

                @pl.when(p < NQ - 1)
                def _():
                    pltpu.async_copy(idx_hbm.at[cid + 4], ibuf.at[b], isem[b])

                def launch_next():
                    pltpu.make_async_copy(idx_hbm.at[cid], ibuf.at[b2], isem[b2]).wait()
                    gathers(b2)
                if b < 2:
                    launch_next()
                else:
                    pl.when(p < NQ - 1)(launch_next)

                tb = tbuf.at[b]
                ub = ubuf.at[b]
                gb = gbuf.at[q]
                pb = pubuf.at[q]

                @plsc.parallel_loop(0, CH, unroll=4)
                def _(e):
                    t8 = tb[e, pl.ds(H, LANES)]
                    u8 = ub[e, pl.ds(H, LANES)]
                    r = t8 - u8
                    rel = r * mask3
                    d = r[0] * r[0] + r[1] * r[1] + r[2] * r[2]
                    wp = t8[3]
                    pb[e, :] = wp * rel
                    for j in range(NJ):
                        sl = pl.ds(LANES * j, LANES)
                        gb[e, sl] = tb[e, sl] + ub[e, sl] + d * w1cs[j]

                pltpu.async_copy(gbuf.at[q], g_hbm.at[pl.ds(e0, CH)], stm[q])
                pltpu.async_copy(pubuf.at[q], shared_pos.at[sbuf.at[q]], scm[q],
                                 add=True)

        for q in (0, 1):
            pltpu.make_async_copy(gbuf.at[q], g_hbm.at[pl.ds(0, CH)], stm[q]).wait()
            pltpu.make_async_copy(pubuf.at[q], shared_pos.at[sbuf.at[q]], scm[q]).wait()

        plsc.subcore_barrier()
        for kk in range(nfull):
            pltpu.sync_copy(shared_pos.at[pl.ds(r0 + CH * kk, CH)], pubuf.at[0])
            pltpu.sync_copy(pubuf.at[0], pacc_hbm.at[c, pl.ds(r0 + CH * kk, CH)])
        if rem:
            pltpu.sync_copy(shared_pos.at[pl.ds(r0 + CH * nfull, rem)],
                            pubuf.at[0, pl.ds(0, rem)])
            pltpu.sync_copy(pubuf.at[0, pl.ds(0, rem)],
                            pacc_hbm.at[c, pl.ds(r0 + CH * nfull, rem)])

    return k(T, U, idxcat, w1c)


def _edge_mlp(g, W2T, b2):
    E_pad, H = g.shape
    D = W2T.shape[1]
    BE = 2048
    assert E_pad % BE == 0

    def body(g_ref, w2_ref, b2_ref, msg_ref):
        h = jax.nn.silu(g_ref[...]).astype(jnp.bfloat16)
        msg_ref[...] = jnp.dot(h, w2_ref[...], preferred_element_type=jnp.float32) + b2_ref[...]

    return pl.pallas_call(
        body,
        grid=(E_pad // BE,),
        in_specs=[
            pl.BlockSpec((BE, H), lambda i: (i, 0)),
            pl.BlockSpec((H, D), lambda i: (0, 0)),
            pl.BlockSpec((1, D), lambda i: (0, 0)),
        ],
        out_specs=pl.BlockSpec((BE, D), lambda i: (i, 0)),
        out_shape=jax.ShapeDtypeStruct((E_pad, D), jnp.float32),
    )(g, W2T, b2)


def _scatter_msg(msg, idxcat, NP):
    E_pad, D = msg.shape
    PWC = E_pad // (NW * CH)
    assert PWC % 2 == 0
    NPAIR = PWC // 2
    RPS = NP // NS
    nfull, rem = RPS // CH, RPS % CH
    mesh = plsc.VectorSubcoreMesh(core_axis_name="c", subcore_axis_name="s")

    @functools.partial(
        pl.kernel,
        mesh=mesh,
        compiler_params=pltpu.CompilerParams(use_tc_tiling_on_sc=False),
        out_type=jax.ShapeDtypeStruct((NC, NP, D), jnp.float32),
        scratch_types=[
            pltpu.VMEM((2, CH), jnp.int32),
            pltpu.VMEM((2, CH, D), jnp.float32),
            pltpu.VMEM_SHARED((NP, D), jnp.float32),
            pltpu.SemaphoreType.DMA,
            pltpu.SemaphoreType.DMA,
            pltpu.SemaphoreType.DMA,
            pltpu.SemaphoreType.DMA,
        ],
    )
    def k(msg_hbm, idx_hbm, xacc_hbm, ibuf, mbuf, shared_x,
          lsem0, lsem1, ssem0, ssem1):
        c = lax.axis_index("c")
        s = lax.axis_index("s")
        wid = s * NC + c
        lsem = (lsem0, lsem1)
        ssem = (ssem0, ssem1)
        zero16 = jnp.zeros((LANES,), jnp.float32)

        @pl.loop(0, CH)
        def _(r):
            for j in range(D // LANES):
                mbuf[0, r, pl.ds(LANES * j, LANES)] = zero16

        r0 = s * RPS
        for kk in range(nfull):
            pltpu.sync_copy(mbuf.at[0], shared_x.at[pl.ds(r0 + CH * kk, CH)])
        if rem:
            pltpu.sync_copy(mbuf.at[0, pl.ds(0, rem)],
                            shared_x.at[pl.ds(r0 + CH * nfull, rem)])
        plsc.subcore_barrier()

        cid0 = wid * PWC

        def loads(cid, b):
            pltpu.async_copy(msg_hbm.at[pl.ds(cid * CH, CH)], mbuf.at[b], lsem[b])
            pltpu.async_copy(idx_hbm.at[cid, 2], ibuf.at[b], lsem[b])

        loads(cid0, 0)

        @pl.loop(0, NPAIR)
        def _(p):
            for b in (0, 1):
                nb = 1 - b
                ch = 2 * p + b
                cid = cid0 + ch

                def drain():
                    pltpu.make_async_copy(mbuf.at[nb], shared_x.at[ibuf.at[nb]],
                                          ssem[nb]).wait()
                if b == 1:
                    drain()
                else:
                    pl.when(p > 0)(drain)

                def prefetch():
                    loads(cid + 1, nb)
                if b == 0:
                    prefetch()
                else:
                    pl.when(p < NPAIR - 1)(prefetch)

                pltpu.make_async_copy(msg_hbm.at[pl.ds(cid * CH, CH)], mbuf.at[b],
                                      lsem[b]).wait()
                pltpu.make_async_copy(idx_hbm.at[cid, 2], ibuf.at[b], lsem[b]).wait()
                pltpu.async_copy(mbuf.at[b], shared_x.at[ibuf.at[b]], ssem[b],
                                 add=True)

        pltpu.make_async_copy(mbuf.at[1], shared_x.at[ibuf.at[1]], ssem[1]).wait()

        plsc.subcore_barrier()
        for kk in range(nfull):
            pltpu.sync_copy(shared_x.at[pl.ds(r0 + CH * kk, CH)], mbuf.at[0])
            pltpu.sync_copy(mbuf.at[0], xacc_hbm.at[c, pl.ds(r0 + CH * kk, CH)])
        if rem:
            pltpu.sync_copy(shared_x.at[pl.ds(r0 + CH * nfull, rem)],
                            mbuf.at[0, pl.ds(0, rem)])
            pltpu.sync_copy(mbuf.at[0, pl.ds(0, rem)],
                            xacc_hbm.at[c, pl.ds(r0 + CH * nfull, rem)])

    return k(msg, idxcat)


def _combine(xacca, xaccb, pacca, paccb, N):
    _, NP, D = xacca.shape
    L = pacca.shape[2]
    BN = 1000
    assert N % BN == 0

    def body(xa_ref, xb_ref, pa_ref, pb_ref, ax_ref, ap_ref):
        ax_ref[...] = (xa_ref[0] + xa_ref[1]) + (xb_ref[0] + xb_ref[1])
        ps = (pa_ref[0] + pa_ref[1]) + (pb_ref[0] + pb_ref[1])
        ap_ref[...] = ps[:, :3]

    return pl.pallas_call(
        body,
        grid=(N // BN,),
        in_specs=[
            pl.BlockSpec((NC, BN, D), lambda i: (0, i, 0)),
            pl.BlockSpec((NC, BN, D), lambda i: (0, i, 0)),
            pl.BlockSpec((NC, BN, L), lambda i: (0, i, 0)),
            pl.BlockSpec((NC, BN, L), lambda i: (0, i, 0)),
        ],
        out_specs=[
            pl.BlockSpec((BN, D), lambda i: (i, 0)),
            pl.BlockSpec((BN, 3), lambda i: (i, 0)),
        ],
        out_shape=[
            jax.ShapeDtypeStruct((N, D), jnp.float32),
            jax.ShapeDtypeStruct((N, 3), jnp.float32),
        ],
    )(xacca, xaccb, pacca, paccb)


def kernel(x, pos, edge_index, W1, b1, W2, b2, W3, b3, W4, b4):
    N, D = x.shape
    E = edge_index.shape[1]
    H = W1.shape[0]

    PWC = -(-E // (NW * CH))
    PWC = -(-PWC // 4) * 4
    E_pad = NW * CH * PWC
    PAD = E_pad - E
    NDUM = 64
    NP = -(-(N + NDUM) // (NS * 8)) * (NS * 8)

    W1aT = W1[:, :D].T
    W1bT = W1[:, D:2 * D].T
    w1c = W1[:, 2 * D]
    W2T = W2.T.astype(jnp.bfloat16)
    W3T = W3.T
    b1r = b1.reshape(1, H)
    b2r = b2.reshape(1, D)
    b3r = b3.reshape(1, H)
    w4row = W4.reshape(1, H)
    b4r = b4.reshape(1, 1)

    xa, xbb, wp = _node_tables(x, W1aT, W1bT, b1r, W3T, b3r, w4row, b4r)

    zpadT = jnp.zeros((N, 12), jnp.float32)
    zpadU = jnp.zeros((N, 13), jnp.float32)
    T = jnp.concatenate([xa, pos, wp, zpadT], axis=1)
    U = jnp.concatenate([xbb, pos, zpadU], axis=1)

    row = edge_index[0]
    col = edge_index[1]
    spread = (jnp.arange(PAD, dtype=jnp.int32) * 97) % N
    rowp = jnp.concatenate([row, spread])
    colg = jnp.concatenate([col, spread])
    cols = jnp.concatenate(
        [col, (N + jnp.arange(PAD, dtype=jnp.int32) % NDUM)])
    idxcat = (jnp.stack([rowp, colg, cols], axis=0)
              .reshape(3, E_pad // CH, CH)
              .transpose(1, 0, 2))

    E_half = E_pad // 2
    M_half = E_half // CH
    idxa, idxb = idxcat[:M_half], idxcat[M_half:]

    ga, pacca = _edge_gather_g(T, U, idxa, w1c, E_half, NP, H)
    gb, paccb = _edge_gather_g(T, U, idxb, w1c, E_half, NP, H)
    msga = _edge_mlp(ga, W2T, b2r)
    msgb = _edge_mlp(gb, W2T, b2r)
    xacca = _scatter_msg(msga, idxa, NP)
    xaccb = _scatter_msg(msgb, idxb, NP)
    aggregated_x, aggregated_pos = _combine(xacca, xaccb, pacca, paccb, N)
    return (aggregated_x, aggregated_pos)


# speedup vs baseline: 8.8077x; 1.0196x over previous
import functools

import jax
import jax.numpy as jnp
from jax import lax
from jax.experimental import pallas as pl
from jax.experimental.pallas import tpu as pltpu
from jax.experimental.pallas import tpu_sc as plsc

NC = 2
NS = 16
NW = NC * NS
LANES = 16
CH = 80


def _node_tables(x, pos, W1aT, W1bT, b1, W3T, b3, w4row, b4, TW):
    N, D = x.shape
    H = W1aT.shape[1]
    BN = 1000
    assert N % BN == 0

    def body(x_ref, pos_ref, w1a_ref, w1b_ref, b1_ref, w3_ref, b3_ref,
             w4_ref, b4_ref, t_ref, u_ref):
        xb = x_ref[...]
        pb = pos_ref[...]
        xa = jnp.dot(xb, w1a_ref[...], preferred_element_type=jnp.float32)
        xbb = jnp.dot(xb, w1b_ref[...], preferred_element_type=jnp.float32) + b1_ref[...]
        h2 = jax.nn.silu(jnp.dot(xb, w3_ref[...], preferred_element_type=jnp.float32) + b3_ref[...])
        wp = jnp.sum(h2 * w4_ref[...], axis=1, keepdims=True) + b4_ref[...]
        zt = jnp.zeros((BN, TW - H - 4), jnp.float32)
        zu = jnp.zeros((BN, TW - H - 3), jnp.float32)
        t_ref[...] = jnp.concatenate([xa, pb, wp, zt], axis=1)
        u_ref[...] = jnp.concatenate([xbb, pb, zu], axis=1)

    return pl.pallas_call(
        body,
        grid=(N // BN,),
        in_specs=[
            pl.BlockSpec((BN, D), lambda i: (i, 0)),
            pl.BlockSpec((BN, 3), lambda i: (i, 0)),
            pl.BlockSpec((D, H), lambda i: (0, 0)),
            pl.BlockSpec((D, H), lambda i: (0, 0)),
            pl.BlockSpec((1, H), lambda i: (0, 0)),
            pl.BlockSpec((D, H), lambda i: (0, 0)),
            pl.BlockSpec((1, H), lambda i: (0, 0)),
            pl.BlockSpec((1, H), lambda i: (0, 0)),
            pl.BlockSpec((1, 1), lambda i: (0, 0)),
        ],
        out_specs=[
            pl.BlockSpec((BN, TW), lambda i: (i, 0)),
            pl.BlockSpec((BN, TW), lambda i: (i, 0)),
        ],
        out_shape=[
            jax.ShapeDtypeStruct((N, TW), jnp.float32),
            jax.ShapeDtypeStruct((N, TW), jnp.float32),
        ],
    )(x, pos, W1aT, W1bT, b1, W3T, b3, w4row, b4)


def _edge_gather_g(T, U, idxcat, w1c, E_pad, NP, H):
    TW = T.shape[1]
    PWC = E_pad // (NW * CH)
    NQ = PWC // 4
    RPS = NP // NS
    NJ = H // LANES
    nfull, rem = RPS // CH, RPS % CH
    mesh = plsc.VectorSubcoreMesh(core_axis_name="c", subcore_axis_name="s")

    @functools.partial(
        pl.kernel,
        mesh=mesh,
        compiler_params=pltpu.CompilerParams(use_tc_tiling_on_sc=False),
        out_type=[
            jax.ShapeDtypeStruct((E_pad, H), jnp.float32),
            jax.ShapeDtypeStruct((NC, NP, LANES), jnp.float32),
        ],
        scratch_types=[
            pltpu.VMEM((4, 3, CH), jnp.int32),
            pltpu.VMEM((2, CH), jnp.int32),
            pltpu.VMEM((4, CH, TW), jnp.float32),
            pltpu.VMEM((4, CH, TW), jnp.float32),
            pltpu.VMEM((2, CH, H), jnp.float32),
            pltpu.VMEM((2, CH, LANES), jnp.float32),
            pltpu.VMEM((H,), jnp.float32),
            pltpu.VMEM_SHARED((NP, LANES), jnp.float32),
            pltpu.SemaphoreType.DMA,
            pltpu.SemaphoreType.DMA,
            pltpu.SemaphoreType.DMA,
            pltpu.SemaphoreType.DMA,
            pltpu.SemaphoreType.DMA,
            pltpu.SemaphoreType.DMA,
            pltpu.SemaphoreType.DMA,
            pltpu.SemaphoreType.DMA,
            pltpu.SemaphoreType.DMA,
            pltpu.SemaphoreType.DMA,
            pltpu.SemaphoreType.DMA,
            pltpu.SemaphoreType.DMA,
        ],
    )
    def k(t_hbm, u_hbm, idx_hbm, w1c_hbm,
          g_hbm, pacc_hbm,
          ibuf, sbuf, tbuf, ubuf, gbuf, pubuf, w1cv, shared_pos,
          isem0, isem1, isem2, isem3, gsm0, gsm1, gsm2, gsm3,
          stm0, stm1, scm0, scm1):
        c = lax.axis_index("c")
        s = lax.axis_index("s")
        wid = s * NC + c
        isem = (isem0, isem1, isem2, isem3)
        gsm = (gsm0, gsm1, gsm2, gsm3)
        stm = (stm0, stm1)
        scm = (scm0, scm1)

        pltpu.sync_copy(w1c_hbm, w1cv)
        w1cs = [w1cv[pl.ds(LANES * j, LANES)] for j in range(NJ)]
        io = lax.iota(jnp.int32, LANES)
        mask3 = jnp.where(io < 3, 1.0, 0.0).astype(jnp.float32)
        zero16 = jnp.zeros((LANES,), jnp.float32)

        @pl.loop(0, CH)
        def _(r):
            pubuf[0, r, :] = zero16

        r0 = s * RPS
        for kk in range(nfull):
            pltpu.sync_copy(pubuf.at[0], shared_pos.at[pl.ds(r0 + CH * kk, CH)])
        if rem:
            pltpu.sync_copy(pubuf.at[0, pl.ds(0, rem)],
                            shared_pos.at[pl.ds(r0 + CH * nfull, rem)])
        plsc.subcore_barrier()

        cid0 = wid * PWC

        def gathers(b):
            pltpu.async_copy(t_hbm.at[ibuf.at[b, 0]], tbuf.at[b], gsm[b])
            pltpu.async_copy(u_hbm.at[ibuf.at[b, 1]], ubuf.at[b], gsm[b])

        pltpu.sync_copy(idx_hbm.at[cid0], ibuf.at[0])
        pltpu.sync_copy(idx_hbm.at[cid0 + 1], ibuf.at[1])
        gathers(0)
        gathers(1)
        pltpu.async_copy(idx_hbm.at[cid0 + 2], ibuf.at[2], isem[2])
        pltpu.async_copy(idx_hbm.at[cid0 + 3], ibuf.at[3], isem[3])

        @pl.loop(0, NQ)
        def _(p):
            for b in range(4):
                q = b % 2
                b2 = (b + 2) % 4
                ch = 4 * p + b
                cid = cid0 + ch
                e0 = cid * CH

                pltpu.make_async_copy(t_hbm.at[ibuf.at[b, 0]], tbuf.at[b], gsm[b]).wait()
                pltpu.make_async_copy(u_hbm.at[ibuf.at[b, 1]], ubuf.at[b], gsm[b]).wait()

                def drain():
                    pltpu.make_async_copy(gbuf.at[q], g_hbm.at[pl.ds(e0, CH)], stm[q]).wait()
                    pltpu.make_async_copy(pubuf.at[q], shared_pos.at[sbuf.at[q]], scm[q]).wait()
                if b < 2:
                    pl.when(p > 0)(drain)
                else:
                    drain()

                for j in range(CH // LANES):
                    sl = pl.ds(LANES * j, LANES)
                    sbuf[q, sl] = ibuf[b, 2, sl]

                @pl.when(p < NQ - 1)
                def _():
                    pltpu.async_copy(idx_hbm.at[cid + 4], ibuf.at[b], isem[b])

                def launch_next():
                    pltpu.make_async_copy(idx_hbm.at[cid], ibuf.at[b2], isem[b2]).wait()
                    gathers(b2)
                if b < 2:
                    launch_next()
                else:
                    pl.when(p < NQ - 1)(launch_next)

                tb = tbuf.at[b]
                ub = ubuf.at[b]
                gb = gbuf.at[q]
                pb = pubuf.at[q]

                @plsc.parallel_loop(0, CH, unroll=4)
                def _(e):
                    t8 = tb[e, pl.ds(H, LANES)]
                    u8 = ub[e, pl.ds(H, LANES)]
                    r = t8 - u8
                    rel = r * mask3
                    d = r[0] * r[0] + r[1] * r[1] + r[2] * r[2]
                    wp = t8[3]
                    pb[e, :] = wp * rel
                    for j in range(NJ):
                        sl = pl.ds(LANES * j, LANES)
                        gb[e, sl] = tb[e, sl] + ub[e, sl] + d * w1cs[j]

                pltpu.async_copy(gbuf.at[q], g_hbm.at[pl.ds(e0, CH)], stm[q])
                pltpu.async_copy(pubuf.at[q], shared_pos.at[sbuf.at[q]], scm[q],
                                 add=True)

        for q in (0, 1):
            pltpu.make_async_copy(gbuf.at[q], g_hbm.at[pl.ds(0, CH)], stm[q]).wait()
            pltpu.make_async_copy(pubuf.at[q], shared_pos.at[sbuf.at[q]], scm[q]).wait()

        plsc.subcore_barrier()
        for kk in range(nfull):
            pltpu.sync_copy(shared_pos.at[pl.ds(r0 + CH * kk, CH)], pubuf.at[0])
            pltpu.sync_copy(pubuf.at[0], pacc_hbm.at[c, pl.ds(r0 + CH * kk, CH)])
        if rem:
            pltpu.sync_copy(shared_pos.at[pl.ds(r0 + CH * nfull, rem)],
                            pubuf.at[0, pl.ds(0, rem)])
            pltpu.sync_copy(pubuf.at[0, pl.ds(0, rem)],
                            pacc_hbm.at[c, pl.ds(r0 + CH * nfull, rem)])

    return k(T, U, idxcat, w1c)


def _edge_mlp(g, W2T, b2):
    E_pad, H = g.shape
    D = W2T.shape[1]
    BE = 2048
    assert E_pad % BE == 0

    def body(g_ref, w2_ref, b2_ref, msg_ref):
        h = jax.nn.silu(g_ref[...]).astype(jnp.bfloat16)
        msg_ref[...] = jnp.dot(h, w2_ref[...], preferred_element_type=jnp.float32) + b2_ref[...]

    return pl.pallas_call(
        body,
        grid=(E_pad // BE,),
        in_specs=[
            pl.BlockSpec((BE, H), lambda i: (i, 0)),
            pl.BlockSpec((H, D), lambda i: (0, 0)),
            pl.BlockSpec((1, D), lambda i: (0, 0)),
        ],
        out_specs=pl.BlockSpec((BE, D), lambda i: (i, 0)),
        out_shape=jax.ShapeDtypeStruct((E_pad, D), jnp.float32),
    )(g, W2T, b2)


def _scatter_msg(msg, idxcat, NP):
    E_pad, D = msg.shape
    PWC = E_pad // (NW * CH)
    assert PWC % 2 == 0
    NPAIR = PWC // 2
    RPS = NP // NS
    nfull, rem = RPS // CH, RPS % CH
    mesh = plsc.VectorSubcoreMesh(core_axis_name="c", subcore_axis_name="s")

    @functools.partial(
        pl.kernel,
        mesh=mesh,
        compiler_params=pltpu.CompilerParams(use_tc_tiling_on_sc=False),
        out_type=jax.ShapeDtypeStruct((NC, NP, D), jnp.float32),
        scratch_types=[
            pltpu.VMEM((2, CH), jnp.int32),
            pltpu.VMEM((2, CH, D), jnp.float32),
            pltpu.VMEM_SHARED((NP, D), jnp.float32),
            pltpu.SemaphoreType.DMA,
            pltpu.SemaphoreType.DMA,
            pltpu.SemaphoreType.DMA,
            pltpu.SemaphoreType.DMA,
        ],
    )
    def k(msg_hbm, idx_hbm, xacc_hbm, ibuf, mbuf, shared_x,
          lsem0, lsem1, ssem0, ssem1):
        c = lax.axis_index("c")
        s = lax.axis_index("s")
        wid = s * NC + c
        lsem = (lsem0, lsem1)
        ssem = (ssem0, ssem1)
        zero16 = jnp.zeros((LANES,), jnp.float32)

        @pl.loop(0, CH)
        def _(r):
            for j in range(D // LANES):
                mbuf[0, r, pl.ds(LANES * j, LANES)] = zero16

        r0 = s * RPS
        for kk in range(nfull):
            pltpu.sync_copy(mbuf.at[0], shared_x.at[pl.ds(r0 + CH * kk, CH)])
        if rem:
            pltpu.sync_copy(mbuf.at[0, pl.ds(0, rem)],
                            shared_x.at[pl.ds(r0 + CH * nfull, rem)])
        plsc.subcore_barrier()

        cid0 = wid * PWC

        def loads(cid, b):
            pltpu.async_copy(msg_hbm.at[pl.ds(cid * CH, CH)], mbuf.at[b], lsem[b])
            pltpu.async_copy(idx_hbm.at[cid, 2], ibuf.at[b], lsem[b])

        loads(cid0, 0)

        @pl.loop(0, NPAIR)
        def _(p):
            for b in (0, 1):
                nb = 1 - b
                ch = 2 * p + b
                cid = cid0 + ch

                def drain():
                    pltpu.make_async_copy(mbuf.at[nb], shared_x.at[ibuf.at[nb]],
                                          ssem[nb]).wait()
                if b == 1:
                    drain()
                else:
                    pl.when(p > 0)(drain)

                def prefetch():
                    loads(cid + 1, nb)
                if b == 0:
                    prefetch()
                else:
                    pl.when(p < NPAIR - 1)(prefetch)

                pltpu.make_async_copy(msg_hbm.at[pl.ds(cid * CH, CH)], mbuf.at[b],
                                      lsem[b]).wait()
                pltpu.make_async_copy(idx_hbm.at[cid, 2], ibuf.at[b], lsem[b]).wait()
                pltpu.async_copy(mbuf.at[b], shared_x.at[ibuf.at[b]], ssem[b],
                                 add=True)

        pltpu.make_async_copy(mbuf.at[1], shared_x.at[ibuf.at[1]], ssem[1]).wait()

        plsc.subcore_barrier()
        for kk in range(nfull):
            pltpu.sync_copy(shared_x.at[pl.ds(r0 + CH * kk, CH)], mbuf.at[0])
            pltpu.sync_copy(mbuf.at[0], xacc_hbm.at[c, pl.ds(r0 + CH * kk, CH)])
        if rem:
            pltpu.sync_copy(shared_x.at[pl.ds(r0 + CH * nfull, rem)],
                            mbuf.at[0, pl.ds(0, rem)])
            pltpu.sync_copy(mbuf.at[0, pl.ds(0, rem)],
                            xacc_hbm.at[c, pl.ds(r0 + CH * nfull, rem)])

    return k(msg, idxcat)


def _combine(xacca, xaccb, pacca, paccb, N):
    _, NP, D = xacca.shape
    L = pacca.shape[2]
    BN = 1000
    assert N % BN == 0

    def body(xa_ref, xb_ref, pa_ref, pb_ref, ax_ref, ap_ref):
        ax_ref[...] = (xa_ref[0] + xa_ref[1]) + (xb_ref[0] + xb_ref[1])
        ps = (pa_ref[0] + pa_ref[1]) + (pb_ref[0] + pb_ref[1])
        ap_ref[...] = ps[:, :3]

    return pl.pallas_call(
        body,
        grid=(N // BN,),
        in_specs=[
            pl.BlockSpec((NC, BN, D), lambda i: (0, i, 0)),
            pl.BlockSpec((NC, BN, D), lambda i: (0, i, 0)),
            pl.BlockSpec((NC, BN, L), lambda i: (0, i, 0)),
            pl.BlockSpec((NC, BN, L), lambda i: (0, i, 0)),
        ],
        out_specs=[
            pl.BlockSpec((BN, D), lambda i: (i, 0)),
            pl.BlockSpec((BN, 3), lambda i: (i, 0)),
        ],
        out_shape=[
            jax.ShapeDtypeStruct((N, D), jnp.float32),
            jax.ShapeDtypeStruct((N, 3), jnp.float32),
        ],
    )(xacca, xaccb, pacca, paccb)


def kernel(x, pos, edge_index, W1, b1, W2, b2, W3, b3, W4, b4):
    N, D = x.shape
    E = edge_index.shape[1]
    H = W1.shape[0]

    PWC = -(-E // (NW * CH))
    PWC = -(-PWC // 8) * 8
    E_pad = NW * CH * PWC
    PAD = E_pad - E
    NDUM = 64
    NP = -(-(N + NDUM) // (NS * 8)) * (NS * 8)

    W1aT = W1[:, :D].T
    W1bT = W1[:, D:2 * D].T
    w1c = W1[:, 2 * D]
    W2T = W2.T.astype(jnp.bfloat16)
    W3T = W3.T
    b1r = b1.reshape(1, H)
    b2r = b2.reshape(1, D)
    b3r = b3.reshape(1, H)
    w4row = W4.reshape(1, H)
    b4r = b4.reshape(1, 1)

    TW = 144
    T, U = _node_tables(x, pos, W1aT, W1bT, b1r, W3T, b3r, w4row, b4r, TW)

    row = edge_index[0]
    col = edge_index[1]
    spread = (jnp.arange(PAD, dtype=jnp.int32) * 97) % N
    rowp = jnp.concatenate([row, spread])
    colg = jnp.concatenate([col, spread])
    cols = jnp.concatenate(
        [col, (N + jnp.arange(PAD, dtype=jnp.int32) % NDUM)])
    idxcat = (jnp.stack([rowp, colg, cols], axis=0)
              .reshape(3, E_pad // CH, CH)
              .transpose(1, 0, 2))

    E_half = E_pad // 2
    M_half = E_half // CH
    idxa, idxb = idxcat[:M_half], idxcat[M_half:]

    ga, pacca = _edge_gather_g(T, U, idxa, w1c, E_half, NP, H)
    gb, paccb = _edge_gather_g(T, U, idxb, w1c, E_half, NP, H)
    msga = _edge_mlp(ga, W2T, b2r)
    msgb = _edge_mlp(gb, W2T, b2r)
    xacca = _scatter_msg(msga, idxa, NP)
    xaccb = _scatter_msg(msgb, idxb, NP)
    aggregated_x, aggregated_pos = _combine(xacca, xaccb, pacca, paccb, N)
    return (aggregated_x, aggregated_pos)


# speedup vs baseline: 9.1583x; 1.0398x over previous
import functools

import jax
import jax.numpy as jnp
from jax import lax
from jax.experimental import pallas as pl
from jax.experimental.pallas import tpu as pltpu
from jax.experimental.pallas import tpu_sc as plsc

NC = 2
NS = 16
NW = NC * NS
LANES = 16
CH = 80


def _node_tables(x, pos, W1aT, W1bT, b1, W3T, b3, w4row, b4, TW):
    N, D = x.shape
    H = W1aT.shape[1]
    BN = 1000
    assert N % BN == 0

    def body(x_ref, pos_ref, w1a_ref, w1b_ref, b1_ref, w3_ref, b3_ref,
             w4_ref, b4_ref, t_ref, u_ref):
        xb = x_ref[...]
        pb = pos_ref[...]
        xa = jnp.dot(xb, w1a_ref[...], preferred_element_type=jnp.float32)
        xbb = jnp.dot(xb, w1b_ref[...], preferred_element_type=jnp.float32) + b1_ref[...]
        h2 = jax.nn.silu(jnp.dot(xb, w3_ref[...], preferred_element_type=jnp.float32) + b3_ref[...])
        wp = jnp.sum(h2 * w4_ref[...], axis=1, keepdims=True) + b4_ref[...]
        zt = jnp.zeros((BN, TW - H - 4), jnp.float32)
        zu = jnp.zeros((BN, TW - H - 3), jnp.float32)
        t_ref[...] = jnp.concatenate([xa, pb, wp, zt], axis=1)
        u_ref[...] = jnp.concatenate([xbb, pb, zu], axis=1)

    return pl.pallas_call(
        body,
        grid=(N // BN,),
        in_specs=[
            pl.BlockSpec((BN, D), lambda i: (i, 0)),
            pl.BlockSpec((BN, 3), lambda i: (i, 0)),
            pl.BlockSpec((D, H), lambda i: (0, 0)),
            pl.BlockSpec((D, H), lambda i: (0, 0)),
            pl.BlockSpec((1, H), lambda i: (0, 0)),
            pl.BlockSpec((D, H), lambda i: (0, 0)),
            pl.BlockSpec((1, H), lambda i: (0, 0)),
            pl.BlockSpec((1, H), lambda i: (0, 0)),
            pl.BlockSpec((1, 1), lambda i: (0, 0)),
        ],
        out_specs=[
            pl.BlockSpec((BN, TW), lambda i: (i, 0)),
            pl.BlockSpec((BN, TW), lambda i: (i, 0)),
        ],
        out_shape=[
            jax.ShapeDtypeStruct((N, TW), jnp.float32),
            jax.ShapeDtypeStruct((N, TW), jnp.float32),
        ],
    )(x, pos, W1aT, W1bT, b1, W3T, b3, w4row, b4)


def _edge_gather_g(T, U, rowp, colg, cols, w1c, E0, E_pad, NP, H):
    TW = T.shape[1]
    PWC = E_pad // (NW * CH)
    NQ = PWC // 4
    RPS = NP // NS
    NJ = H // LANES
    nfull, rem = RPS // CH, RPS % CH
    mesh = plsc.VectorSubcoreMesh(core_axis_name="c", subcore_axis_name="s")

    @functools.partial(
        pl.kernel,
        mesh=mesh,
        compiler_params=pltpu.CompilerParams(use_tc_tiling_on_sc=False),
        out_type=[
            jax.ShapeDtypeStruct((E_pad, H), jnp.float32),
            jax.ShapeDtypeStruct((NC, NP, LANES), jnp.float32),
        ],
        scratch_types=[
            pltpu.VMEM((4, 3, CH), jnp.int32),
            pltpu.VMEM((2, CH), jnp.int32),
            pltpu.VMEM((4, CH, TW), jnp.float32),
            pltpu.VMEM((4, CH, TW), jnp.float32),
            pltpu.VMEM((2, CH, H), jnp.float32),
            pltpu.VMEM((2, CH, LANES), jnp.float32),
            pltpu.VMEM((H,), jnp.float32),
            pltpu.VMEM_SHARED((NP, LANES), jnp.float32),
            pltpu.SemaphoreType.DMA,
            pltpu.SemaphoreType.DMA,
            pltpu.SemaphoreType.DMA,
            pltpu.SemaphoreType.DMA,
            pltpu.SemaphoreType.DMA,
            pltpu.SemaphoreType.DMA,
            pltpu.SemaphoreType.DMA,
            pltpu.SemaphoreType.DMA,
            pltpu.SemaphoreType.DMA,
            pltpu.SemaphoreType.DMA,
            pltpu.SemaphoreType.DMA,
            pltpu.SemaphoreType.DMA,
        ],
    )
    def k(t_hbm, u_hbm, rowp_hbm, colg_hbm, cols_hbm, w1c_hbm,
          g_hbm, pacc_hbm,
          ibuf, sbuf, tbuf, ubuf, gbuf, pubuf, w1cv, shared_pos,
          isem0, isem1, isem2, isem3, gsm0, gsm1, gsm2, gsm3,
          stm0, stm1, scm0, scm1):
        c = lax.axis_index("c")
        s = lax.axis_index("s")
        wid = s * NC + c
        isem = (isem0, isem1, isem2, isem3)
        gsm = (gsm0, gsm1, gsm2, gsm3)
        stm = (stm0, stm1)
        scm = (scm0, scm1)

        pltpu.sync_copy(w1c_hbm, w1cv)
        w1cs = [w1cv[pl.ds(LANES * j, LANES)] for j in range(NJ)]
        io = lax.iota(jnp.int32, LANES)
        mask3 = jnp.where(io < 3, 1.0, 0.0).astype(jnp.float32)
        zero16 = jnp.zeros((LANES,), jnp.float32)

        @pl.loop(0, CH)
        def _(r):
            pubuf[0, r, :] = zero16

        r0 = s * RPS
        for kk in range(nfull):
            pltpu.sync_copy(pubuf.at[0], shared_pos.at[pl.ds(r0 + CH * kk, CH)])
        if rem:
            pltpu.sync_copy(pubuf.at[0, pl.ds(0, rem)],
                            shared_pos.at[pl.ds(r0 + CH * nfull, rem)])
        plsc.subcore_barrier()

        cid0 = wid * PWC

        def gathers(b):
            pltpu.async_copy(t_hbm.at[ibuf.at[b, 0]], tbuf.at[b], gsm[b])
            pltpu.async_copy(u_hbm.at[ibuf.at[b, 1]], ubuf.at[b], gsm[b])

        def idx_load(cid, b):
            e0 = cid * CH + E0
            pltpu.async_copy(rowp_hbm.at[pl.ds(e0, CH)], ibuf.at[b, 0], isem[b])
            pltpu.async_copy(colg_hbm.at[pl.ds(e0, CH)], ibuf.at[b, 1], isem[b])
            pltpu.async_copy(cols_hbm.at[pl.ds(e0, CH)], ibuf.at[b, 2], isem[b])

        def idx_wait(cid, b):
            e0 = cid * CH + E0
            pltpu.make_async_copy(rowp_hbm.at[pl.ds(e0, CH)], ibuf.at[b, 0], isem[b]).wait()
            pltpu.make_async_copy(colg_hbm.at[pl.ds(e0, CH)], ibuf.at[b, 1], isem[b]).wait()
            pltpu.make_async_copy(cols_hbm.at[pl.ds(e0, CH)], ibuf.at[b, 2], isem[b]).wait()

        idx_load(cid0, 0)
        idx_load(cid0 + 1, 1)
        idx_wait(cid0, 0)
        idx_wait(cid0 + 1, 1)
        gathers(0)
        gathers(1)
        idx_load(cid0 + 2, 2)
        idx_load(cid0 + 3, 3)

        @pl.loop(0, NQ)
        def _(p):
            for b in range(4):
                q = b % 2
                b2 = (b + 2) % 4
                ch = 4 * p + b
                cid = cid0 + ch
                e0 = cid * CH

                pltpu.make_async_copy(t_hbm.at[ibuf.at[b, 0]], tbuf.at[b], gsm[b]).wait()
                pltpu.make_async_copy(u_hbm.at[ibuf.at[b, 1]], ubuf.at[b], gsm[b]).wait()

                def drain():
                    pltpu.make_async_copy(gbuf.at[q], g_hbm.at[pl.ds(e0, CH)], stm[q]).wait()
                    pltpu.make_async_copy(pubuf.at[q], shared_pos.at[sbuf.at[q]], scm[q]).wait()
                if b < 2:
                    pl.when(p > 0)(drain)
                else:
                    drain()

                for j in range(CH // LANES):
                    sl = pl.ds(LANES * j, LANES)
                    sbuf[q, sl] = ibuf[b, 2, sl]

                @pl.when(p < NQ - 1)
                def _():
                    idx_load(cid + 4, b)

                def launch_next():
                    idx_wait(cid + 2, b2)
                    gathers(b2)
                if b < 2:
                    launch_next()
                else:
                    pl.when(p < NQ - 1)(launch_next)

                tb = tbuf.at[b]
                ub = ubuf.at[b]
                gb = gbuf.at[q]
                pb = pubuf.at[q]

                @plsc.parallel_loop(0, CH, unroll=4)
                def _(e):
                    t8 = tb[e, pl.ds(H, LANES)]
                    u8 = ub[e, pl.ds(H, LANES)]
                    r = t8 - u8
                    rel = r * mask3
                    d = r[0] * r[0] + r[1] * r[1] + r[2] * r[2]
                    wp = t8[3]
                    pb[e, :] = wp * rel
                    for j in range(NJ):
                        sl = pl.ds(LANES * j, LANES)
                        gb[e, sl] = tb[e, sl] + ub[e, sl] + d * w1cs[j]

                pltpu.async_copy(gbuf.at[q], g_hbm.at[pl.ds(e0, CH)], stm[q])
                pltpu.async_copy(pubuf.at[q], shared_pos.at[sbuf.at[q]], scm[q],
                                 add=True)

        for q in (0, 1):
            pltpu.make_async_copy(gbuf.at[q], g_hbm.at[pl.ds(0, CH)], stm[q]).wait()
            pltpu.make_async_copy(pubuf.at[q], shared_pos.at[sbuf.at[q]], scm[q]).wait()

        plsc.subcore_barrier()
        for kk in range(nfull):
            pltpu.sync_copy(shared_pos.at[pl.ds(r0 + CH * kk, CH)], pubuf.at[0])
            pltpu.sync_copy(pubuf.at[0], pacc_hbm.at[c, pl.ds(r0 + CH * kk, CH)])
        if rem:
            pltpu.sync_copy(shared_pos.at[pl.ds(r0 + CH * nfull, rem)],
                            pubuf.at[0, pl.ds(0, rem)])
            pltpu.sync_copy(pubuf.at[0, pl.ds(0, rem)],
                            pacc_hbm.at[c, pl.ds(r0 + CH * nfull, rem)])

    return k(T, U, rowp, colg, cols, w1c)


def _edge_mlp(g, W2T, b2):
    E_pad, H = g.shape
    D = W2T.shape[1]
    BE = 2048
    assert E_pad % BE == 0

    def body(g_ref, w2_ref, b2_ref, msg_ref):
        h = jax.nn.silu(g_ref[...]).astype(jnp.bfloat16)
        msg_ref[...] = jnp.dot(h, w2_ref[...], preferred_element_type=jnp.float32) + b2_ref[...]

    return pl.pallas_call(
        body,
        grid=(E_pad // BE,),
        in_specs=[
            pl.BlockSpec((BE, H), lambda i: (i, 0)),
            pl.BlockSpec((H, D), lambda i: (0, 0)),
            pl.BlockSpec((1, D), lambda i: (0, 0)),
        ],
        out_specs=pl.BlockSpec((BE, D), lambda i: (i, 0)),
        out_shape=jax.ShapeDtypeStruct((E_pad, D), jnp.float32),
    )(g, W2T, b2)


def _scatter_msg(msg, cols, E0, NP):
    E_pad, D = msg.shape
    PWC = E_pad // (NW * CH)
    assert PWC % 2 == 0
    NPAIR = PWC // 2
    RPS = NP // NS
    nfull, rem = RPS // CH, RPS % CH
    mesh = plsc.VectorSubcoreMesh(core_axis_name="c", subcore_axis_name="s")

    @functools.partial(
        pl.kernel,
        mesh=mesh,
        compiler_params=pltpu.CompilerParams(use_tc_tiling_on_sc=False),
        out_type=jax.ShapeDtypeStruct((NC, NP, D), jnp.float32),
        scratch_types=[
            pltpu.VMEM((2, CH), jnp.int32),
            pltpu.VMEM((2, CH, D), jnp.float32),
            pltpu.VMEM_SHARED((NP, D), jnp.float32),
            pltpu.SemaphoreType.DMA,
            pltpu.SemaphoreType.DMA,
            pltpu.SemaphoreType.DMA,
            pltpu.SemaphoreType.DMA,
        ],
    )
    def k(msg_hbm, cols_hbm, xacc_hbm, ibuf, mbuf, shared_x,
          lsem0, lsem1, ssem0, ssem1):
        c = lax.axis_index("c")
        s = lax.axis_index("s")
        wid = s * NC + c
        lsem = (lsem0, lsem1)
        ssem = (ssem0, ssem1)
        zero16 = jnp.zeros((LANES,), jnp.float32)

        @pl.loop(0, CH)
        def _(r):
            for j in range(D // LANES):
                mbuf[0, r, pl.ds(LANES * j, LANES)] = zero16

        r0 = s * RPS
        for kk in range(nfull):
            pltpu.sync_copy(mbuf.at[0], shared_x.at[pl.ds(r0 + CH * kk, CH)])
        if rem:
            pltpu.sync_copy(mbuf.at[0, pl.ds(0, rem)],
                            shared_x.at[pl.ds(r0 + CH * nfull, rem)])
        plsc.subcore_barrier()

        cid0 = wid * PWC

        def loads(cid, b):
            pltpu.async_copy(msg_hbm.at[pl.ds(cid * CH, CH)], mbuf.at[b], lsem[b])
            pltpu.async_copy(cols_hbm.at[pl.ds(cid * CH + E0, CH)], ibuf.at[b], lsem[b])

        loads(cid0, 0)

        @pl.loop(0, NPAIR)
        def _(p):
            for b in (0, 1):
                nb = 1 - b
                ch = 2 * p + b
                cid = cid0 + ch

                def drain():
                    pltpu.make_async_copy(mbuf.at[nb], shared_x.at[ibuf.at[nb]],
                                          ssem[nb]).wait()
                if b == 1:
                    drain()
                else:
                    pl.when(p > 0)(drain)

                def prefetch():
                    loads(cid + 1, nb)
                if b == 0:
                    prefetch()
                else:
                    pl.when(p < NPAIR - 1)(prefetch)

                pltpu.make_async_copy(msg_hbm.at[pl.ds(cid * CH, CH)], mbuf.at[b],
                                      lsem[b]).wait()
                pltpu.make_async_copy(cols_hbm.at[pl.ds(cid * CH + E0, CH)],
                                      ibuf.at[b], lsem[b]).wait()
                pltpu.async_copy(mbuf.at[b], shared_x.at[ibuf.at[b]], ssem[b],
                                 add=True)

        pltpu.make_async_copy(mbuf.at[1], shared_x.at[ibuf.at[1]], ssem[1]).wait()

        plsc.subcore_barrier()
        for kk in range(nfull):
            pltpu.sync_copy(shared_x.at[pl.ds(r0 + CH * kk, CH)], mbuf.at[0])
            pltpu.sync_copy(mbuf.at[0], xacc_hbm.at[c, pl.ds(r0 + CH * kk, CH)])
        if rem:
            pltpu.sync_copy(shared_x.at[pl.ds(r0 + CH * nfull, rem)],
                            mbuf.at[0, pl.ds(0, rem)])
            pltpu.sync_copy(mbuf.at[0, pl.ds(0, rem)],
                            xacc_hbm.at[c, pl.ds(r0 + CH * nfull, rem)])

    return k(msg, cols)


def _combine(xacca, xaccb, pacca, paccb, N):
    _, NP, D = xacca.shape
    L = pacca.shape[2]
    BN = 1000
    assert N % BN == 0

    def body(xa_ref, xb_ref, pa_ref, pb_ref, ax_ref, ap_ref):
        ax_ref[...] = (xa_ref[0] + xa_ref[1]) + (xb_ref[0] + xb_ref[1])
        ps = (pa_ref[0] + pa_ref[1]) + (pb_ref[0] + pb_ref[1])
        ap_ref[...] = ps[:, :3]

    return pl.pallas_call(
        body,
        grid=(N // BN,),
        in_specs=[
            pl.BlockSpec((NC, BN, D), lambda i: (0, i, 0)),
            pl.BlockSpec((NC, BN, D), lambda i: (0, i, 0)),
            pl.BlockSpec((NC, BN, L), lambda i: (0, i, 0)),
            pl.BlockSpec((NC, BN, L), lambda i: (0, i, 0)),
        ],
        out_specs=[
            pl.BlockSpec((BN, D), lambda i: (i, 0)),
            pl.BlockSpec((BN, 3), lambda i: (i, 0)),
        ],
        out_shape=[
            jax.ShapeDtypeStruct((N, D), jnp.float32),
            jax.ShapeDtypeStruct((N, 3), jnp.float32),
        ],
    )(xacca, xaccb, pacca, paccb)


def kernel(x, pos, edge_index, W1, b1, W2, b2, W3, b3, W4, b4):
    N, D = x.shape
    E = edge_index.shape[1]
    H = W1.shape[0]

    PWC = -(-E // (NW * CH))
    PWC = -(-PWC // 8) * 8
    E_pad = NW * CH * PWC
    PAD = E_pad - E
    NDUM = 64
    NP = -(-(N + NDUM) // (NS * 8)) * (NS * 8)

    W1aT = W1[:, :D].T
    W1bT = W1[:, D:2 * D].T
    w1c = W1[:, 2 * D]
    W2T = W2.T.astype(jnp.bfloat16)
    W3T = W3.T
    b1r = b1.reshape(1, H)
    b2r = b2.reshape(1, D)
    b3r = b3.reshape(1, H)
    w4row = W4.reshape(1, H)
    b4r = b4.reshape(1, 1)

    TW = 144
    T, U = _node_tables(x, pos, W1aT, W1bT, b1r, W3T, b3r, w4row, b4r, TW)

    row = edge_index[0]
    col = edge_index[1]
    spread = (jnp.arange(PAD, dtype=jnp.int32) * 97) % N
    rowp = jnp.concatenate([row, spread])
    colg = jnp.concatenate([col, spread])
    cols = jnp.concatenate(
        [col, (N + jnp.arange(PAD, dtype=jnp.int32) % NDUM)])
    E_half = E_pad // 2

    ga, pacca = _edge_gather_g(T, U, rowp, colg, cols, w1c, 0, E_half, NP, H)
    gb, paccb = _edge_gather_g(T, U, rowp, colg, cols, w1c, E_half, E_half, NP, H)
    msga = _edge_mlp(ga, W2T, b2r)
    msgb = _edge_mlp(gb, W2T, b2r)
    xacca = _scatter_msg(msga, cols, 0, NP)
    xaccb = _scatter_msg(msgb, cols, E_half, NP)
    aggregated_x, aggregated_pos = _combine(xacca, xaccb, pacca, paccb, N)
    return (aggregated_x, aggregated_pos)
